# baseline (speedup 1.0000x reference)
import functools

import jax
import jax.numpy as jnp
from jax.experimental import pallas as pl

N_SKILL = 100000
N_JOB = 10000
N_RES = 10000
D = 128
H = 8
E = 320000


def _leaky(x, s=0.01):
    return jnp.where(x >= 0, x, s * x)


def _mlp_body(comb_ref, w1_ref, b1_ref, w2_ref, b2_ref, w3_ref, b3_ref, out_ref):
    comb = comb_ref[...]
    h1 = _leaky(jnp.dot(comb, w1_ref[...], preferred_element_type=jnp.float32) + b1_ref[...])
    h2 = _leaky(jnp.dot(h1, w2_ref[...], preferred_element_type=jnp.float32) + b2_ref[...])
    score = jnp.sum(h2 * w3_ref[...].reshape(1, D), axis=1) + b3_ref[0]
    out_ref[...] = score


def _mlp(comb, W1, b1, W2, b2, W3, b3):
    n = comb.shape[0]
    return pl.pallas_call(
        _mlp_body,
        out_shape=jax.ShapeDtypeStruct((n,), jnp.float32),
    )(comb, W1, b1, W2, b2, W3, b3)


def kernel(x_skill, x_job, x_resume, edge_index_sj, edge_index_sr, W_rel, b_rel, W_root, W_src, W_dst, att_src, att_dst, b_gat, W1, b1, W2, b2, W3, b3):
    src_j = edge_index_sj[0]
    dst_j = edge_index_sj[1]
    agg = jax.ops.segment_sum(jnp.take(x_skill, src_j, axis=0), dst_j, num_segments=N_JOB)
    x_job_out = agg @ W_rel + b_rel + x_job @ W_root
    src_r = edge_index_sr[0]
    dst_r = edge_index_sr[1]
    xs = (x_skill @ W_src).reshape(N_SKILL, H, D)
    xd = (x_resume @ W_dst).reshape(N_RES, H, D)
    a_src = jnp.sum(xs * att_src[None, :, :], axis=-1)
    a_dst = jnp.sum(xd * att_dst[None, :, :], axis=-1)
    alpha = _leaky(jnp.take(a_src, src_r, axis=0) + jnp.take(a_dst, dst_r, axis=0), 0.2)
    amax = jax.ops.segment_max(alpha, dst_r, num_segments=N_RES)
    amax = jnp.where(jnp.isfinite(amax), amax, 0.0)
    ex = jnp.exp(alpha - jnp.take(amax, dst_r, axis=0))
    den = jax.ops.segment_sum(ex, dst_r, num_segments=N_RES)
    attn = ex / (jnp.take(den, dst_r, axis=0) + 1e-16)
    msg = jnp.take(xs, src_r, axis=0) * attn[:, :, None]
    out = jax.ops.segment_sum(msg, dst_r, num_segments=N_RES)
    x_res_out = jnp.mean(out, axis=1) + b_gat
    xj = _leaky(x_job_out)
    xr = _leaky(x_res_out)
    comb = jnp.concatenate([xj, xr], axis=1)
    score = _mlp(comb, W1, b1, W2, b2, W3, b3)
    return (score, attn)


# speedup vs baseline: 7.8047x; 7.8047x over previous
import functools

import jax
import jax.numpy as jnp
from jax import lax
from jax.experimental import pallas as pl
from jax.experimental.pallas import tpu as pltpu
from jax.experimental.pallas import tpu_sc as plsc

N_SKILL = 100000
N_JOB = 10000
N_RES = 10000
D = 128
H = 8
E = 320000

NW = 32
R_A = 320
PAD_N = R_A * NW
FIRE = 128
SCHUNK = 4000
R_C = 80
NPASS = 4
SCHUNK_C = 2000
ECHUNK = 80

_SC_MESH = dict(
    mesh=plsc.VectorSubcoreMesh(core_axis_name="c", subcore_axis_name="s"),
    compiler_params=pltpu.CompilerParams(needs_layout_passes=False, use_tc_tiling_on_sc=False),
)


def _leaky(x, s=0.01):
    return jnp.where(x >= 0, x, s * x)


def _wid():
    return lax.axis_index("s") * 2 + lax.axis_index("c")


def _zero_rows(ref, nrows, ncols):
    def _z(i, carry):
        for kk in range(ncols // 16):
            ref[i, pl.ds(kk * 16, 16)] = jnp.zeros((16,), jnp.float32)
        return carry
    lax.fori_loop(0, nrows, _z, 0)


@functools.partial(
    pl.kernel,
    out_type=jax.ShapeDtypeStruct((PAD_N, D), jnp.float32),
    scratch_types=[
        pltpu.VMEM((SCHUNK,), jnp.int32),
        pltpu.VMEM((SCHUNK,), jnp.int32),
        pltpu.VMEM((SCHUNK + 160,), jnp.int32),
        pltpu.VMEM((SCHUNK + 160,), jnp.int32),
        pltpu.VMEM((FIRE,), jnp.int32),
        pltpu.VMEM((FIRE, D), jnp.float32),
        pltpu.VMEM((R_A + 8, D), jnp.float32),
        pltpu.SemaphoreType.DMA,
    ],
    **_SC_MESH,
)
def _gc_kernel(x_hbm, src_hbm, dst_hbm, out_hbm,
               srcb, dstb, pend_src, pend_row, fire_idx, xbuf, acc, sem):
    lo = _wid() * R_A
    iota = lax.iota(jnp.int32, 16)
    _zero_rows(acc, R_A + 8, D)

    def do_fire(off):
        for t in range(FIRE // 16):
            fire_idx[pl.ds(t * 16, 16)] = pend_src[pl.ds(off + t * 16, 16)]
        pltpu.async_copy(x_hbm.at[fire_idx], xbuf, sem).wait()

        def edge(j, carry):
            rowv = plsc.load_gather(pend_row, [jnp.full((16,), off + j, jnp.int32)])
            jv = jnp.full((16,), j, jnp.int32)
            for k in range(8):
                col = iota + (k * 16)
                vals = plsc.load_gather(xbuf, [jv, col])
                plsc.addupdate_scatter(acc, [rowv, col], vals)
            return carry
        lax.fori_loop(0, FIRE, edge, 0)

    def scan_step(ci, cnt):
        base = ci * SCHUNK
        pltpu.sync_copy(src_hbm.at[pl.ds(base, SCHUNK)], srcb)
        pltpu.sync_copy(dst_hbm.at[pl.ds(base, SCHUNK)], dstb)

        def group(gi, cnt):
            sv = srcb[pl.ds(gi * 16, 16)]
            dv = dstb[pl.ds(gi * 16, 16)]
            m = (dv >= lo) & (dv < lo + R_A)
            cs = plsc.cumsum(jnp.where(m, 1, 0))
            pos = cnt + cs - 1
            plsc.store_scatter(pend_src, [pos], sv, mask=m)
            plsc.store_scatter(pend_row, [pos], dv - lo, mask=m)
            return cnt + cs[15]

        cnt = lax.fori_loop(0, SCHUNK // 16, group, cnt)
        n_full = cnt // FIRE

        def fire_j(j, carry):
            do_fire(j * FIRE)
            return carry
        lax.fori_loop(0, n_full, fire_j, 0)

        ro = n_full * FIRE
        for t in range(8):
            pend_src[pl.ds(t * 16, 16)] = pend_src[pl.ds(ro + t * 16, 16)]
            pend_row[pl.ds(t * 16, 16)] = pend_row[pl.ds(ro + t * 16, 16)]
        return cnt - ro

    cnt = lax.fori_loop(0, E // SCHUNK, scan_step, jnp.int32(0))

    for t in range(8):
        pend_src[pl.ds(cnt + t * 16, 16)] = jnp.zeros((16,), jnp.int32)
        pend_row[pl.ds(cnt + t * 16, 16)] = jnp.full((16,), R_A, jnp.int32)
    do_fire(0)

    pltpu.sync_copy(acc.at[pl.ds(0, R_A)], out_hbm.at[pl.ds(lo, R_A)])


@functools.partial(
    pl.kernel,
    out_type=jax.ShapeDtypeStruct((PAD_N, 16), jnp.float32),
    scratch_types=[
        pltpu.VMEM((SCHUNK,), jnp.int32),
        pltpu.VMEM((SCHUNK,), jnp.int32),
        pltpu.VMEM((SCHUNK + 160,), jnp.int32),
        pltpu.VMEM((SCHUNK + 160,), jnp.int32),
        pltpu.VMEM((FIRE,), jnp.int32),
        pltpu.VMEM((FIRE, 16), jnp.float32),
        pltpu.VMEM((R_A, 16), jnp.float32),
        pltpu.VMEM((R_A + 8, 16), jnp.float32),
        pltpu.SemaphoreType.DMA,
    ],
    **_SC_MESH,
)
def _den_kernel(asrc_hbm, adst_hbm, src_hbm, dst_hbm, out_hbm,
                srcb, dstb, pend_src, pend_row, fire_idx, abuf, adst_own, acc, sem):
    lo = _wid() * R_A
    iota = lax.iota(jnp.int32, 16)
    _zero_rows(acc, R_A + 8, 16)
    pltpu.sync_copy(adst_hbm.at[pl.ds(lo, R_A)], adst_own)

    def do_fire(off):
        for t in range(FIRE // 16):
            fire_idx[pl.ds(t * 16, 16)] = pend_src[pl.ds(off + t * 16, 16)]
        pltpu.async_copy(asrc_hbm.at[fire_idx], abuf, sem).wait()

        def edge(j, carry):
            rowv = plsc.load_gather(pend_row, [jnp.full((16,), off + j, jnp.int32)])
            jv = jnp.full((16,), j, jnp.int32)
            a_s = plsc.load_gather(abuf, [jv, iota])
            a_d = plsc.load_gather(adst_own, [rowv, iota])
            al = a_s + a_d
            al = jnp.where(al >= 0, al, 0.2 * al)
            ex = jnp.exp(al)
            plsc.addupdate_scatter(acc, [rowv, iota], ex)
            return carry
        lax.fori_loop(0, FIRE, edge, 0)

    def scan_step(ci, cnt):
        base = ci * SCHUNK
        pltpu.sync_copy(src_hbm.at[pl.ds(base, SCHUNK)], srcb)
        pltpu.sync_copy(dst_hbm.at[pl.ds(base, SCHUNK)], dstb)

        def group(gi, cnt):
            sv = srcb[pl.ds(gi * 16, 16)]
            dv = dstb[pl.ds(gi * 16, 16)]
            m = (dv >= lo) & (dv < lo + R_A)
            cs = plsc.cumsum(jnp.where(m, 1, 0))
            pos = cnt + cs - 1
            plsc.store_scatter(pend_src, [pos], sv, mask=m)
            plsc.store_scatter(pend_row, [pos], dv - lo, mask=m)
            return cnt + cs[15]

        cnt = lax.fori_loop(0, SCHUNK // 16, group, cnt)
        n_full = cnt // FIRE

        def fire_j(j, carry):
            do_fire(j * FIRE)
            return carry
        lax.fori_loop(0, n_full, fire_j, 0)

        ro = n_full * FIRE
        for t in range(8):
            pend_src[pl.ds(t * 16, 16)] = pend_src[pl.ds(ro + t * 16, 16)]
            pend_row[pl.ds(t * 16, 16)] = pend_row[pl.ds(ro + t * 16, 16)]
        return cnt - ro

    cnt = lax.fori_loop(0, E // SCHUNK, scan_step, jnp.int32(0))

    for t in range(8):
        pend_src[pl.ds(cnt + t * 16, 16)] = jnp.zeros((16,), jnp.int32)
        pend_row[pl.ds(cnt + t * 16, 16)] = jnp.full((16,), R_A, jnp.int32)
    do_fire(0)

    pltpu.sync_copy(acc.at[pl.ds(0, R_A)], out_hbm.at[pl.ds(lo, R_A)])


@functools.partial(
    pl.kernel,
    out_type=jax.ShapeDtypeStruct((E, 16), jnp.float32),
    scratch_types=[
        pltpu.VMEM((ECHUNK,), jnp.int32),
        pltpu.VMEM((ECHUNK,), jnp.int32),
        pltpu.VMEM((ECHUNK, 16), jnp.float32),
        pltpu.VMEM((ECHUNK, 16), jnp.float32),
        pltpu.VMEM((ECHUNK, 16), jnp.float32),
        pltpu.VMEM((ECHUNK, 16), jnp.float32),
        pltpu.SemaphoreType.DMA,
        pltpu.SemaphoreType.DMA,
        pltpu.SemaphoreType.DMA,
    ],
    **_SC_MESH,
)
def _attn_kernel(asrc_hbm, adst_hbm, den_hbm, src_hbm, dst_hbm, out_hbm,
                 srcc, dstc, gs, gd, gn, outb, sem1, sem2, sem3):
    iota = lax.iota(jnp.int32, 16)
    e0 = _wid() * (E // NW)

    def chunk(ki, carry):
        base = e0 + ki * ECHUNK
        pltpu.sync_copy(src_hbm.at[pl.ds(base, ECHUNK)], srcc)
        pltpu.sync_copy(dst_hbm.at[pl.ds(base, ECHUNK)], dstc)
        ca = pltpu.async_copy(asrc_hbm.at[srcc], gs, sem1)
        cb = pltpu.async_copy(adst_hbm.at[dstc], gd, sem2)
        cc = pltpu.async_copy(den_hbm.at[dstc], gn, sem3)
        ca.wait()
        cb.wait()
        cc.wait()

        def edge(j, carry):
            jv = jnp.full((16,), j, jnp.int32)
            a_s = plsc.load_gather(gs, [jv, iota])
            a_d = plsc.load_gather(gd, [jv, iota])
            dn = plsc.load_gather(gn, [jv, iota])
            al = a_s + a_d
            al = jnp.where(al >= 0, al, 0.2 * al)
            ex = jnp.exp(al)
            outb[j, pl.ds(0, 16)] = ex / (dn + 1e-16)
            return carry
        lax.fori_loop(0, ECHUNK, edge, 0)
        pltpu.sync_copy(outb, out_hbm.at[pl.ds(base, ECHUNK)])
        return carry

    lax.fori_loop(0, (E // NW) // ECHUNK, chunk, 0)


@functools.partial(
    pl.kernel,
    out_type=jax.ShapeDtypeStruct((PAD_N, H * D), jnp.float32),
    scratch_types=[
        pltpu.VMEM((SCHUNK_C,), jnp.int32),
        pltpu.VMEM((SCHUNK_C,), jnp.int32),
        pltpu.VMEM((SCHUNK_C + 160,), jnp.int32),
        pltpu.VMEM((SCHUNK_C + 160,), jnp.int32),
        pltpu.VMEM((SCHUNK_C + 160,), jnp.int32),
        pltpu.VMEM((FIRE,), jnp.int32),
        pltpu.VMEM((FIRE,), jnp.int32),
        pltpu.VMEM((FIRE, D), jnp.float32),
        pltpu.VMEM((FIRE, 16), jnp.float32),
        pltpu.VMEM((R_C + 4, H * D), jnp.float32),
        pltpu.SemaphoreType.DMA,
        pltpu.SemaphoreType.DMA,
    ],
    **_SC_MESH,
)
def _gatagg_kernel(x_hbm, attn_hbm, src_hbm, dst_hbm, out_hbm,
                   srcb, dstb, pend_src, pend_row, pend_id,
                   fire_idx, fire_id, xbuf, abuf, acc, sem1, sem2):
    w = _wid()
    iota = lax.iota(jnp.int32, 16)

    for p in range(NPASS):
        lo = (p * NW + w) * R_C
        _zero_rows(acc, R_C + 4, H * D)

        def do_fire(off):
            for t in range(FIRE // 16):
                fire_idx[pl.ds(t * 16, 16)] = pend_src[pl.ds(off + t * 16, 16)]
                fire_id[pl.ds(t * 16, 16)] = pend_id[pl.ds(off + t * 16, 16)]
            ca = pltpu.async_copy(x_hbm.at[fire_idx], xbuf, sem1)
            cb = pltpu.async_copy(attn_hbm.at[fire_id], abuf, sem2)
            ca.wait()
            cb.wait()

            def edge(j, carry):
                rowv = plsc.load_gather(pend_row, [jnp.full((16,), off + j, jnp.int32)])
                jv = jnp.full((16,), j, jnp.int32)
                coefs = [plsc.load_gather(abuf, [jv, jnp.full((16,), h, jnp.int32)])
                         for h in range(H)]
                for k in range(8):
                    vals = plsc.load_gather(xbuf, [jv, iota + (k * 16)])
                    for h in range(H):
                        col = iota + (h * D + k * 16)
                        plsc.addupdate_scatter(acc, [rowv, col], vals * coefs[h])
                return carry
            lax.fori_loop(0, FIRE, edge, 0)

        def scan_step(ci, cnt):
            base = ci * SCHUNK_C
            pltpu.sync_copy(src_hbm.at[pl.ds(base, SCHUNK_C)], srcb)
            pltpu.sync_copy(dst_hbm.at[pl.ds(base, SCHUNK_C)], dstb)

            def group(gi, cnt):
                sv = srcb[pl.ds(gi * 16, 16)]
                dv = dstb[pl.ds(gi * 16, 16)]
                ev = base + gi * 16 + iota
                m = (dv >= lo) & (dv < lo + R_C)
                cs = plsc.cumsum(jnp.where(m, 1, 0))
                pos = cnt + cs - 1
                plsc.store_scatter(pend_src, [pos], sv, mask=m)
                plsc.store_scatter(pend_row, [pos], dv - lo, mask=m)
                plsc.store_scatter(pend_id, [pos], ev, mask=m)
                return cnt + cs[15]

            cnt = lax.fori_loop(0, SCHUNK_C // 16, group, cnt)
            n_full = cnt // FIRE

            def fire_j(j, carry):
                do_fire(j * FIRE)
                return carry
            lax.fori_loop(0, n_full, fire_j, 0)

            ro = n_full * FIRE
            for t in range(8):
                pend_src[pl.ds(t * 16, 16)] = pend_src[pl.ds(ro + t * 16, 16)]
                pend_row[pl.ds(t * 16, 16)] = pend_row[pl.ds(ro + t * 16, 16)]
                pend_id[pl.ds(t * 16, 16)] = pend_id[pl.ds(ro + t * 16, 16)]
            return cnt - ro

        cnt = lax.fori_loop(0, E // SCHUNK_C, scan_step, jnp.int32(0))

        for t in range(8):
            pend_src[pl.ds(cnt + t * 16, 16)] = jnp.zeros((16,), jnp.int32)
            pend_row[pl.ds(cnt + t * 16, 16)] = jnp.full((16,), R_C, jnp.int32)
            pend_id[pl.ds(cnt + t * 16, 16)] = jnp.zeros((16,), jnp.int32)
        do_fire(0)

        pltpu.sync_copy(acc.at[pl.ds(0, R_C)], out_hbm.at[pl.ds(lo, R_C)])


def _proj_body(x_ref, w_ref, att_ref, out_ref):
    w3 = w_ref[...].reshape(D, H, D)
    v = jnp.sum(w3 * att_ref[...][None, :, :], axis=-1)
    vpad = jnp.concatenate([v, jnp.zeros((D, 16 - H), jnp.float32)], axis=1)
    out_ref[...] = jnp.dot(x_ref[...], vpad, preferred_element_type=jnp.float32)


def _proj(x, W, att, blk):
    n = x.shape[0]
    return pl.pallas_call(
        _proj_body,
        grid=(n // blk,),
        in_specs=[
            pl.BlockSpec((blk, D), lambda i: (i, 0)),
            pl.BlockSpec((D, H * D), lambda i: (0, 0)),
            pl.BlockSpec((H, D), lambda i: (0, 0)),
        ],
        out_specs=pl.BlockSpec((blk, 16), lambda i: (i, 0)),
        out_shape=jax.ShapeDtypeStruct((n, 16), jnp.float32),
    )(x, W, att)


def _job_body(agg_ref, xj_ref, wrel_ref, wroot_ref, b_ref, out_ref):
    r = (jnp.dot(agg_ref[...], wrel_ref[...], preferred_element_type=jnp.float32)
         + jnp.dot(xj_ref[...], wroot_ref[...], preferred_element_type=jnp.float32)
         + b_ref[...])
    out_ref[...] = _leaky(r)


def _job_out(agg, x_job, W_rel, W_root, b_rel):
    blk = 2000
    return pl.pallas_call(
        _job_body,
        grid=(N_JOB // blk,),
        in_specs=[
            pl.BlockSpec((blk, D), lambda i: (i, 0)),
            pl.BlockSpec((blk, D), lambda i: (i, 0)),
            pl.BlockSpec((D, D), lambda i: (0, 0)),
            pl.BlockSpec((D, D), lambda i: (0, 0)),
            pl.BlockSpec((D,), lambda i: (0,)),
        ],
        out_specs=pl.BlockSpec((blk, D), lambda i: (i, 0)),
        out_shape=jax.ShapeDtypeStruct((N_JOB, D), jnp.float32),
    )(agg, x_job, W_rel, W_root, b_rel)


def _gatout_body(c_ref, ws_ref, b_ref, out_ref):
    r = jnp.dot(c_ref[...], ws_ref[...], preferred_element_type=jnp.float32)
    out_ref[...] = _leaky(r * (1.0 / H) + b_ref[...])


def _gat_out(C, W_stack, b_gat):
    blk = 2000
    return pl.pallas_call(
        _gatout_body,
        grid=(N_RES // blk,),
        in_specs=[
            pl.BlockSpec((blk, H * D), lambda i: (i, 0)),
            pl.BlockSpec((H * D, D), lambda i: (0, 0)),
            pl.BlockSpec((D,), lambda i: (0,)),
        ],
        out_specs=pl.BlockSpec((blk, D), lambda i: (i, 0)),
        out_shape=jax.ShapeDtypeStruct((N_RES, D), jnp.float32),
    )(C, W_stack, b_gat)


def _mlp_body(comb_ref, w1_ref, b1_ref, w2_ref, b2_ref, w3_ref, b3_ref, out_ref):
    comb = comb_ref[...]
    h1 = _leaky(jnp.dot(comb, w1_ref[...], preferred_element_type=jnp.float32) + b1_ref[...])
    h2 = _leaky(jnp.dot(h1, w2_ref[...], preferred_element_type=jnp.float32) + b2_ref[...])
    score = jnp.sum(h2 * w3_ref[...].reshape(1, D), axis=1) + b3_ref[0]
    out_ref[...] = score


def _mlp(comb, W1, b1, W2, b2, W3, b3):
    n = comb.shape[0]
    return pl.pallas_call(
        _mlp_body,
        out_shape=jax.ShapeDtypeStruct((n,), jnp.float32),
    )(comb, W1, b1, W2, b2, W3, b3)


def kernel(x_skill, x_job, x_resume, edge_index_sj, edge_index_sr, W_rel, b_rel, W_root, W_src, W_dst, att_src, att_dst, b_gat, W1, b1, W2, b2, W3, b3):
    src_j = edge_index_sj[0]
    dst_j = edge_index_sj[1]
    src_r = edge_index_sr[0]
    dst_r = edge_index_sr[1]

    agg = _gc_kernel(x_skill, src_j, dst_j)[:N_JOB]
    xj = _job_out(agg, x_job, W_rel, W_root, b_rel)

    a_src = _proj(x_skill, W_src, att_src, 2000)
    a_dst = _proj(x_resume, W_dst, att_dst, 2000)
    a_dst_pad = jnp.concatenate(
        [a_dst, jnp.zeros((PAD_N - N_RES, 16), jnp.float32)], axis=0)

    den = _den_kernel(a_src, a_dst_pad, src_r, dst_r)
    attn_pad = _attn_kernel(a_src, a_dst_pad, den, src_r, dst_r)
    attn = attn_pad[:, :H]

    C = _gatagg_kernel(x_skill, attn_pad, src_r, dst_r)[:N_RES]
    W_stack = W_src.reshape(D, H, D).transpose(1, 0, 2).reshape(H * D, D)
    xr = _gat_out(C, W_stack, b_gat)

    comb = jnp.concatenate([xj, xr], axis=1)
    score = _mlp(comb, W1, b1, W2, b2, W3, b3)
    return (score, attn)


# speedup vs baseline: 8.1934x; 1.0498x over previous
import functools

import jax
import jax.numpy as jnp
from jax import lax
from jax.experimental import pallas as pl
from jax.experimental.pallas import tpu as pltpu
from jax.experimental.pallas import tpu_sc as plsc

N_SKILL = 100000
N_JOB = 10000
N_RES = 10000
D = 128
H = 8
E = 320000

NW = 32
R_A = 320
PAD_N = R_A * NW
FIRE = 128
SCHUNK = 8000
R_C = 80
NPASS = 4
SCHUNK_C = 2000
ECHUNK = 80

_SC_MESH = dict(
    mesh=plsc.VectorSubcoreMesh(core_axis_name="c", subcore_axis_name="s"),
    compiler_params=pltpu.CompilerParams(
        needs_layout_passes=False, use_tc_tiling_on_sc=False),
)


def _leaky(x, s=0.01):
    return jnp.where(x >= 0, x, s * x)


def _wid():
    return lax.axis_index("s") * 2 + lax.axis_index("c")


def _zero_flat(ref, n):
    def _z(i, carry):
        for kk in range(8):
            ref[pl.ds(i * 128 + kk * 16, 16)] = jnp.zeros((16,), jnp.float32)
        return carry
    lax.fori_loop(0, n // 128, _z, 0)


def _splat(x):
    return jnp.full((16,), x, jnp.int32)


@functools.partial(
    pl.kernel,
    out_type=jax.ShapeDtypeStruct((PAD_N * D,), jnp.float32),
    scratch_types=[
        pltpu.VMEM((SCHUNK,), jnp.int32),
        pltpu.VMEM((SCHUNK,), jnp.int32),
        pltpu.VMEM((SCHUNK + 160,), jnp.int32),
        pltpu.VMEM((SCHUNK + 160,), jnp.int32),
        pltpu.VMEM((FIRE,), jnp.int32),
        pltpu.VMEM((FIRE, D), jnp.float32),
        pltpu.VMEM(((R_A + 8) * D,), jnp.float32),
        pltpu.SemaphoreType.DMA,
    ],
    **_SC_MESH,
)
def _gc_kernel(x_hbm, src_hbm, dst_hbm, out_hbm,
               srcb, dstb, pend_src, pend_row, fire_idx, xbuf, acc, sem):
    lo = _wid() * R_A
    iota = lax.iota(jnp.int32, 16)
    _zero_flat(acc, (R_A + 8) * D)

    def do_fire_full(off):
        for t in range(FIRE // 16):
            fire_idx[pl.ds(t * 16, 16)] = pend_src[pl.ds(off + t * 16, 16)]
        pltpu.async_copy(x_hbm.at[fire_idx], xbuf, sem).wait()

        def edge2(jj, carry):
            for u in range(2):
                j = jj * 2 + u
                rowv = plsc.load_gather(pend_row, [_splat(off + j)])
                rbase = rowv * D + iota
                for k in range(8):
                    vals = xbuf[j, pl.ds(k * 16, 16)]
                    plsc.addupdate_scatter(acc, [rbase + (k * 16)], vals)
            return carry
        lax.fori_loop(0, FIRE // 2, edge2, 0)

    def scan_step(ci, cnt):
        base = ci * SCHUNK
        pltpu.sync_copy(src_hbm.at[pl.ds(base, SCHUNK)], srcb)
        pltpu.sync_copy(dst_hbm.at[pl.ds(base, SCHUNK)], dstb)

        def group2(gg, cnt):
            for u in range(2):
                gi = gg * 2 + u
                sv = srcb[pl.ds(gi * 16, 16)]
                dv = dstb[pl.ds(gi * 16, 16)]
                m = (dv >= lo) & (dv < lo + R_A)
                cs = plsc.cumsum(jnp.where(m, 1, 0))
                pos = cnt + cs - 1
                plsc.store_scatter(pend_src, [pos], sv, mask=m)
                plsc.store_scatter(pend_row, [pos], dv - lo, mask=m)
                cnt = cnt + cs[15]
            return cnt

        cnt = lax.fori_loop(0, SCHUNK // 32, group2, cnt)
        n_full = cnt // FIRE

        def fire_j(j, carry):
            do_fire_full(j * FIRE)
            return carry
        lax.fori_loop(0, n_full, fire_j, 0)

        ro = n_full * FIRE
        for t in range(8):
            pend_src[pl.ds(t * 16, 16)] = pend_src[pl.ds(ro + t * 16, 16)]
            pend_row[pl.ds(t * 16, 16)] = pend_row[pl.ds(ro + t * 16, 16)]
        return cnt - ro

    cnt = lax.fori_loop(0, E // SCHUNK, scan_step, jnp.int32(0))

    for t in range(8):
        pend_src[pl.ds(cnt + t * 16, 16)] = jnp.zeros((16,), jnp.int32)
        pend_row[pl.ds(cnt + t * 16, 16)] = jnp.full((16,), R_A, jnp.int32)
    do_fire_full(0)

    pltpu.sync_copy(acc.at[pl.ds(0, R_A * D)], out_hbm.at[pl.ds(lo * D, R_A * D)])


@functools.partial(
    pl.kernel,
    out_type=jax.ShapeDtypeStruct((PAD_N * 16,), jnp.float32),
    scratch_types=[
        pltpu.VMEM((SCHUNK,), jnp.int32),
        pltpu.VMEM((SCHUNK,), jnp.int32),
        pltpu.VMEM((SCHUNK + 160,), jnp.int32),
        pltpu.VMEM((SCHUNK + 160,), jnp.int32),
        pltpu.VMEM((FIRE,), jnp.int32),
        pltpu.VMEM((FIRE, 16), jnp.float32),
        pltpu.VMEM((R_A * 16,), jnp.float32),
        pltpu.VMEM(((R_A + 8) * 16,), jnp.float32),
        pltpu.SemaphoreType.DMA,
    ],
    **_SC_MESH,
)
def _den_kernel(asrc_hbm, adstf_hbm, src_hbm, dst_hbm, out_hbm,
                srcb, dstb, pend_src, pend_row, fire_idx, abuf, adst_own, acc, sem):
    lo = _wid() * R_A
    iota = lax.iota(jnp.int32, 16)
    _zero_flat(acc, (R_A + 8) * 16)
    pltpu.sync_copy(adstf_hbm.at[pl.ds(lo * 16, R_A * 16)], adst_own)

    def do_fire(off):
        for t in range(FIRE // 16):
            fire_idx[pl.ds(t * 16, 16)] = pend_src[pl.ds(off + t * 16, 16)]
        pltpu.async_copy(asrc_hbm.at[fire_idx], abuf, sem).wait()

        def edge2(jj, carry):
            for u in range(2):
                j = jj * 2 + u
                rowv = plsc.load_gather(pend_row, [_splat(off + j)])
                r16 = rowv * 16 + iota
                a_s = abuf[j, pl.ds(0, 16)]
                a_d = plsc.load_gather(adst_own, [r16])
                al = a_s + a_d
                al = jnp.where(al >= 0, al, 0.2 * al)
                plsc.addupdate_scatter(acc, [r16], jnp.exp(al))
            return carry
        lax.fori_loop(0, FIRE // 2, edge2, 0)

    def scan_step(ci, cnt):
        base = ci * SCHUNK
        pltpu.sync_copy(src_hbm.at[pl.ds(base, SCHUNK)], srcb)
        pltpu.sync_copy(dst_hbm.at[pl.ds(base, SCHUNK)], dstb)

        def group2(gg, cnt):
            for u in range(2):
                gi = gg * 2 + u
                sv = srcb[pl.ds(gi * 16, 16)]
                dv = dstb[pl.ds(gi * 16, 16)]
                m = (dv >= lo) & (dv < lo + R_A)
                cs = plsc.cumsum(jnp.where(m, 1, 0))
                pos = cnt + cs - 1
                plsc.store_scatter(pend_src, [pos], sv, mask=m)
                plsc.store_scatter(pend_row, [pos], dv - lo, mask=m)
                cnt = cnt + cs[15]
            return cnt

        cnt = lax.fori_loop(0, SCHUNK // 32, group2, cnt)
        n_full = cnt // FIRE

        def fire_j(j, carry):
            do_fire(j * FIRE)
            return carry
        lax.fori_loop(0, n_full, fire_j, 0)

        ro = n_full * FIRE
        for t in range(8):
            pend_src[pl.ds(t * 16, 16)] = pend_src[pl.ds(ro + t * 16, 16)]
            pend_row[pl.ds(t * 16, 16)] = pend_row[pl.ds(ro + t * 16, 16)]
        return cnt - ro

    cnt = lax.fori_loop(0, E // SCHUNK, scan_step, jnp.int32(0))

    for t in range(8):
        pend_src[pl.ds(cnt + t * 16, 16)] = jnp.zeros((16,), jnp.int32)
        pend_row[pl.ds(cnt + t * 16, 16)] = jnp.full((16,), R_A, jnp.int32)
    do_fire(0)

    pltpu.sync_copy(acc.at[pl.ds(0, R_A * 16)], out_hbm.at[pl.ds(lo * 16, R_A * 16)])


@functools.partial(
    pl.kernel,
    out_type=jax.ShapeDtypeStruct((E, 16), jnp.float32),
    scratch_types=[
        pltpu.VMEM((ECHUNK,), jnp.int32),
        pltpu.VMEM((ECHUNK,), jnp.int32),
        pltpu.VMEM((ECHUNK, 16), jnp.float32),
        pltpu.VMEM((ECHUNK, 16), jnp.float32),
        pltpu.VMEM((ECHUNK, 16), jnp.float32),
        pltpu.VMEM((ECHUNK, 16), jnp.float32),
        pltpu.SemaphoreType.DMA,
        pltpu.SemaphoreType.DMA,
        pltpu.SemaphoreType.DMA,
    ],
    **_SC_MESH,
)
def _attn_kernel(asrc_hbm, adst_hbm, den_hbm, src_hbm, dst_hbm, out_hbm,
                 srcc, dstc, gs, gd, gn, outb, sem1, sem2, sem3):
    e0 = _wid() * (E // NW)

    def chunk(ki, carry):
        base = e0 + ki * ECHUNK
        pltpu.sync_copy(src_hbm.at[pl.ds(base, ECHUNK)], srcc)
        pltpu.sync_copy(dst_hbm.at[pl.ds(base, ECHUNK)], dstc)
        ca = pltpu.async_copy(asrc_hbm.at[srcc], gs, sem1)
        cb = pltpu.async_copy(adst_hbm.at[dstc], gd, sem2)
        cc = pltpu.async_copy(den_hbm.at[dstc], gn, sem3)
        ca.wait()
        cb.wait()
        cc.wait()

        def edge(j, carry):
            a_s = gs[j, pl.ds(0, 16)]
            a_d = gd[j, pl.ds(0, 16)]
            dn = gn[j, pl.ds(0, 16)]
            al = a_s + a_d
            al = jnp.where(al >= 0, al, 0.2 * al)
            outb[j, pl.ds(0, 16)] = jnp.exp(al) / (dn + 1e-16)
            return carry
        lax.fori_loop(0, ECHUNK, edge, 0)
        pltpu.sync_copy(outb, out_hbm.at[pl.ds(base, ECHUNK)])
        return carry

    lax.fori_loop(0, (E // NW) // ECHUNK, chunk, 0)


@functools.partial(
    pl.kernel,
    out_type=jax.ShapeDtypeStruct((PAD_N * H * D,), jnp.float32),
    scratch_types=[
        pltpu.VMEM((SCHUNK_C,), jnp.int32),
        pltpu.VMEM((SCHUNK_C,), jnp.int32),
        pltpu.VMEM((SCHUNK_C + 160,), jnp.int32),
        pltpu.VMEM((SCHUNK_C + 160,), jnp.int32),
        pltpu.VMEM((SCHUNK_C + 160,), jnp.int32),
        pltpu.VMEM((FIRE,), jnp.int32),
        pltpu.VMEM((FIRE,), jnp.int32),
        pltpu.VMEM((FIRE, D), jnp.float32),
        pltpu.VMEM((FIRE, 16), jnp.float32),
        pltpu.VMEM(((R_C + 4) * H * D,), jnp.float32),
        pltpu.SemaphoreType.DMA,
        pltpu.SemaphoreType.DMA,
    ],
    **_SC_MESH,
)
def _gatagg_kernel(x_hbm, attn_hbm, src_hbm, dst_hbm, out_hbm,
                   srcb, dstb, pend_src, pend_row, pend_id,
                   fire_idx, fire_id, xbuf, abuf, acc, sem1, sem2):
    w = _wid()
    iota = lax.iota(jnp.int32, 16)
    HD = H * D

    for p in range(NPASS):
        lo = (p * NW + w) * R_C
        _zero_flat(acc, (R_C + 4) * HD)

        def do_fire(off):
            for t in range(FIRE // 16):
                fire_idx[pl.ds(t * 16, 16)] = pend_src[pl.ds(off + t * 16, 16)]
                fire_id[pl.ds(t * 16, 16)] = pend_id[pl.ds(off + t * 16, 16)]
            ca = pltpu.async_copy(x_hbm.at[fire_idx], xbuf, sem1)
            cb = pltpu.async_copy(attn_hbm.at[fire_id], abuf, sem2)
            ca.wait()
            cb.wait()

            def edge(j, carry):
                rowv = plsc.load_gather(pend_row, [_splat(off + j)])
                rbase = rowv * HD + iota
                jv = _splat(j)
                coefs = [plsc.load_gather(abuf, [jv, _splat(h)]) for h in range(H)]
                for k in range(8):
                    vals = xbuf[j, pl.ds(k * 16, 16)]
                    for h in range(H):
                        plsc.addupdate_scatter(
                            acc, [rbase + (h * D + k * 16)], vals * coefs[h])
                return carry
            lax.fori_loop(0, FIRE, edge, 0)

        def scan_step(ci, cnt):
            base = ci * SCHUNK_C
            pltpu.sync_copy(src_hbm.at[pl.ds(base, SCHUNK_C)], srcb)
            pltpu.sync_copy(dst_hbm.at[pl.ds(base, SCHUNK_C)], dstb)

            def group2(gg, cnt):
                for u in range(2):
                    gi = gg * 2 + u
                    sv = srcb[pl.ds(gi * 16, 16)]
                    dv = dstb[pl.ds(gi * 16, 16)]
                    ev = base + gi * 16 + iota
                    m = (dv >= lo) & (dv < lo + R_C)
                    cs = plsc.cumsum(jnp.where(m, 1, 0))
                    pos = cnt + cs - 1
                    plsc.store_scatter(pend_src, [pos], sv, mask=m)
                    plsc.store_scatter(pend_row, [pos], dv - lo, mask=m)
                    plsc.store_scatter(pend_id, [pos], ev, mask=m)
                    cnt = cnt + cs[15]
                return cnt

            cnt = lax.fori_loop(0, SCHUNK_C // 32, group2, cnt)
            n_full = cnt // FIRE

            def fire_j(j, carry):
                do_fire(j * FIRE)
                return carry
            lax.fori_loop(0, n_full, fire_j, 0)

            ro = n_full * FIRE
            for t in range(8):
                pend_src[pl.ds(t * 16, 16)] = pend_src[pl.ds(ro + t * 16, 16)]
                pend_row[pl.ds(t * 16, 16)] = pend_row[pl.ds(ro + t * 16, 16)]
                pend_id[pl.ds(t * 16, 16)] = pend_id[pl.ds(ro + t * 16, 16)]
            return cnt - ro

        cnt = lax.fori_loop(0, E // SCHUNK_C, scan_step, jnp.int32(0))

        for t in range(8):
            pend_src[pl.ds(cnt + t * 16, 16)] = jnp.zeros((16,), jnp.int32)
            pend_row[pl.ds(cnt + t * 16, 16)] = jnp.full((16,), R_C, jnp.int32)
            pend_id[pl.ds(cnt + t * 16, 16)] = jnp.zeros((16,), jnp.int32)
        do_fire(0)

        pltpu.sync_copy(acc.at[pl.ds(0, R_C * HD)],
                        out_hbm.at[pl.ds(lo * HD, R_C * HD)])


def _proj_body(x_ref, w_ref, att_ref, out_ref):
    w3 = w_ref[...].reshape(D, H, D)
    v = jnp.sum(w3 * att_ref[...][None, :, :], axis=-1)
    vpad = jnp.concatenate([v, jnp.zeros((D, 16 - H), jnp.float32)], axis=1)
    out_ref[...] = jnp.dot(x_ref[...], vpad, preferred_element_type=jnp.float32)


def _proj(x, W, att, blk):
    n = x.shape[0]
    return pl.pallas_call(
        _proj_body,
        grid=(n // blk,),
        in_specs=[
            pl.BlockSpec((blk, D), lambda i: (i, 0)),
            pl.BlockSpec((D, H * D), lambda i: (0, 0)),
            pl.BlockSpec((H, D), lambda i: (0, 0)),
        ],
        out_specs=pl.BlockSpec((blk, 16), lambda i: (i, 0)),
        out_shape=jax.ShapeDtypeStruct((n, 16), jnp.float32),
    )(x, W, att)


def _job_body(agg_ref, xj_ref, wrel_ref, wroot_ref, b_ref, out_ref):
    r = (jnp.dot(agg_ref[...], wrel_ref[...], preferred_element_type=jnp.float32)
         + jnp.dot(xj_ref[...], wroot_ref[...], preferred_element_type=jnp.float32)
         + b_ref[...])
    out_ref[...] = _leaky(r)


def _job_out(agg, x_job, W_rel, W_root, b_rel):
    blk = 2000
    return pl.pallas_call(
        _job_body,
        grid=(N_JOB // blk,),
        in_specs=[
            pl.BlockSpec((blk, D), lambda i: (i, 0)),
            pl.BlockSpec((blk, D), lambda i: (i, 0)),
            pl.BlockSpec((D, D), lambda i: (0, 0)),
            pl.BlockSpec((D, D), lambda i: (0, 0)),
            pl.BlockSpec((D,), lambda i: (0,)),
        ],
        out_specs=pl.BlockSpec((blk, D), lambda i: (i, 0)),
        out_shape=jax.ShapeDtypeStruct((N_JOB, D), jnp.float32),
    )(agg, x_job, W_rel, W_root, b_rel)


def _gatout_body(c_ref, ws_ref, b_ref, out_ref):
    r = jnp.dot(c_ref[...], ws_ref[...], preferred_element_type=jnp.float32)
    out_ref[...] = _leaky(r * (1.0 / H) + b_ref[...])


def _gat_out(C, W_stack, b_gat):
    blk = 2000
    return pl.pallas_call(
        _gatout_body,
        grid=(N_RES // blk,),
        in_specs=[
            pl.BlockSpec((blk, H * D), lambda i: (i, 0)),
            pl.BlockSpec((H * D, D), lambda i: (0, 0)),
            pl.BlockSpec((D,), lambda i: (0,)),
        ],
        out_specs=pl.BlockSpec((blk, D), lambda i: (i, 0)),
        out_shape=jax.ShapeDtypeStruct((N_RES, D), jnp.float32),
    )(C, W_stack, b_gat)


def _mlp_body(comb_ref, w1_ref, b1_ref, w2_ref, b2_ref, w3_ref, b3_ref, out_ref):
    comb = comb_ref[...]
    h1 = _leaky(jnp.dot(comb, w1_ref[...], preferred_element_type=jnp.float32) + b1_ref[...])
    h2 = _leaky(jnp.dot(h1, w2_ref[...], preferred_element_type=jnp.float32) + b2_ref[...])
    score = jnp.sum(h2 * w3_ref[...].reshape(1, D), axis=1) + b3_ref[0]
    out_ref[...] = score


def _mlp(comb, W1, b1, W2, b2, W3, b3):
    n = comb.shape[0]
    return pl.pallas_call(
        _mlp_body,
        out_shape=jax.ShapeDtypeStruct((n,), jnp.float32),
    )(comb, W1, b1, W2, b2, W3, b3)


def kernel(x_skill, x_job, x_resume, edge_index_sj, edge_index_sr, W_rel, b_rel, W_root, W_src, W_dst, att_src, att_dst, b_gat, W1, b1, W2, b2, W3, b3):
    src_j = edge_index_sj[0]
    dst_j = edge_index_sj[1]
    src_r = edge_index_sr[0]
    dst_r = edge_index_sr[1]

    agg = _gc_kernel(x_skill, src_j, dst_j).reshape(PAD_N, D)[:N_JOB]
    xj = _job_out(agg, x_job, W_rel, W_root, b_rel)

    a_src = _proj(x_skill, W_src, att_src, 2000)
    a_dst = _proj(x_resume, W_dst, att_dst, 2000)
    a_dst_pad = jnp.concatenate(
        [a_dst, jnp.zeros((PAD_N - N_RES, 16), jnp.float32)], axis=0)

    den = _den_kernel(a_src, a_dst_pad.reshape(-1), src_r, dst_r).reshape(PAD_N, 16)
    attn_pad = _attn_kernel(a_src, a_dst_pad, den, src_r, dst_r)
    attn = attn_pad[:, :H]

    C = _gatagg_kernel(x_skill, attn_pad, src_r, dst_r).reshape(PAD_N, H * D)[:N_RES]
    W_stack = W_src.reshape(D, H, D).transpose(1, 0, 2).reshape(H * D, D)
    xr = _gat_out(C, W_stack, b_gat)

    comb = jnp.concatenate([xj, xr], axis=1)
    score = _mlp(comb, W1, b1, W2, b2, W3, b3)
    return (score, attn)


# speedup vs baseline: 11.5749x; 1.4127x over previous
import functools

import jax
import jax.numpy as jnp
from jax import lax
from jax.experimental import pallas as pl
from jax.experimental.pallas import tpu as pltpu
from jax.experimental.pallas import tpu_sc as plsc

N_SKILL = 100000
N_JOB = 10000
N_RES = 10000
D = 128
H = 8
E = 320000

NW = 32
R_A = 320
PAD_N = R_A * NW
FIRE = 128
SCHUNK = 8000
R_C = 80
NPASS = 4
SCHUNK_C = 2000
ECHUNK = 80

_SC_MESH = dict(
    mesh=plsc.VectorSubcoreMesh(core_axis_name="c", subcore_axis_name="s"),
    compiler_params=pltpu.CompilerParams(
        needs_layout_passes=False, use_tc_tiling_on_sc=False),
)


def _leaky(x, s=0.01):
    return jnp.where(x >= 0, x, s * x)


def _wid():
    return lax.axis_index("s") * 2 + lax.axis_index("c")


def _zero_flat(ref, n):
    def _z(i, carry):
        for kk in range(8):
            ref[pl.ds(i * 128 + kk * 16, 16)] = jnp.zeros((16,), jnp.float32)
        return carry
    lax.fori_loop(0, n // 128, _z, 0)


def _splat(x):
    return jnp.full((16,), x, jnp.int32)


@functools.partial(
    pl.kernel,
    out_type=jax.ShapeDtypeStruct((PAD_N * D,), jnp.float32),
    scratch_types=[
        pltpu.VMEM((SCHUNK,), jnp.int32),
        pltpu.VMEM((SCHUNK,), jnp.int32),
        pltpu.VMEM((SCHUNK + 160,), jnp.int32),
        pltpu.VMEM((SCHUNK + 160,), jnp.int32),
        pltpu.VMEM((FIRE,), jnp.int32),
        pltpu.VMEM((FIRE, D), jnp.float32),
        pltpu.VMEM(((R_A + 8) * D,), jnp.float32),
        pltpu.SemaphoreType.DMA,
    ],
    **_SC_MESH,
)
def _gc_kernel(x_hbm, src_hbm, dst_hbm, out_hbm,
               srcb, dstb, pend_src, pend_row, fire_idx, xbuf, acc, sem):
    lo = _wid() * R_A
    iota = lax.iota(jnp.int32, 16)
    _zero_flat(acc, (R_A + 8) * D)

    def do_fire_full(off):
        for t in range(FIRE // 16):
            fire_idx[pl.ds(t * 16, 16)] = pend_src[pl.ds(off + t * 16, 16)]
        pltpu.async_copy(x_hbm.at[fire_idx], xbuf, sem).wait()

        def edge(j):
            rowv = plsc.load_gather(pend_row, [_splat(off + j)])
            rbase = rowv * D + iota
            for k in range(8):
                vals = xbuf[j, pl.ds(k * 16, 16)]
                plsc.addupdate_scatter(acc, [rbase + (k * 16)], vals)
        plsc.parallel_loop(0, FIRE, unroll=2)(edge)

    def scan_step(ci, cnt):
        base = ci * SCHUNK
        pltpu.sync_copy(src_hbm.at[pl.ds(base, SCHUNK)], srcb)
        pltpu.sync_copy(dst_hbm.at[pl.ds(base, SCHUNK)], dstb)

        def group(gi, cnt):
            sv = srcb[pl.ds(gi * 16, 16)]
            dv = dstb[pl.ds(gi * 16, 16)]
            m = (dv >= lo) & (dv < lo + R_A)
            cs = plsc.cumsum(jnp.where(m, 1, 0))
            pos = cnt + cs - 1
            plsc.store_scatter(pend_src, [pos], sv, mask=m)
            plsc.store_scatter(pend_row, [pos], dv - lo, mask=m)
            return cnt + cs[15]

        cnt = plsc.parallel_loop(0, SCHUNK // 16, unroll=4, carry=cnt)(group)
        n_full = cnt // FIRE

        def fire_j(j, carry):
            do_fire_full(j * FIRE)
            return carry
        lax.fori_loop(0, n_full, fire_j, 0)

        ro = n_full * FIRE
        for t in range(8):
            pend_src[pl.ds(t * 16, 16)] = pend_src[pl.ds(ro + t * 16, 16)]
            pend_row[pl.ds(t * 16, 16)] = pend_row[pl.ds(ro + t * 16, 16)]
        return cnt - ro

    cnt = lax.fori_loop(0, E // SCHUNK, scan_step, jnp.int32(0))

    for t in range(8):
        pend_src[pl.ds(cnt + t * 16, 16)] = jnp.zeros((16,), jnp.int32)
        pend_row[pl.ds(cnt + t * 16, 16)] = jnp.full((16,), R_A, jnp.int32)
    do_fire_full(0)

    pltpu.sync_copy(acc.at[pl.ds(0, R_A * D)], out_hbm.at[pl.ds(lo * D, R_A * D)])


@functools.partial(
    pl.kernel,
    out_type=jax.ShapeDtypeStruct((PAD_N * 16,), jnp.float32),
    scratch_types=[
        pltpu.VMEM((SCHUNK,), jnp.int32),
        pltpu.VMEM((SCHUNK,), jnp.int32),
        pltpu.VMEM((SCHUNK + 160,), jnp.int32),
        pltpu.VMEM((SCHUNK + 160,), jnp.int32),
        pltpu.VMEM((FIRE,), jnp.int32),
        pltpu.VMEM((FIRE, 16), jnp.float32),
        pltpu.VMEM((R_A * 16,), jnp.float32),
        pltpu.VMEM(((R_A + 8) * 16,), jnp.float32),
        pltpu.SemaphoreType.DMA,
    ],
    **_SC_MESH,
)
def _den_kernel(asrc_hbm, adstf_hbm, src_hbm, dst_hbm, out_hbm,
                srcb, dstb, pend_src, pend_row, fire_idx, abuf, adst_own, acc, sem):
    lo = _wid() * R_A
    iota = lax.iota(jnp.int32, 16)
    _zero_flat(acc, (R_A + 8) * 16)
    pltpu.sync_copy(adstf_hbm.at[pl.ds(lo * 16, R_A * 16)], adst_own)

    def do_fire(off):
        for t in range(FIRE // 16):
            fire_idx[pl.ds(t * 16, 16)] = pend_src[pl.ds(off + t * 16, 16)]
        pltpu.async_copy(asrc_hbm.at[fire_idx], abuf, sem).wait()

        def edge(j):
            rowv = plsc.load_gather(pend_row, [_splat(off + j)])
            r16 = rowv * 16 + iota
            a_s = abuf[j, pl.ds(0, 16)]
            a_d = plsc.load_gather(adst_own, [r16])
            al = a_s + a_d
            al = jnp.where(al >= 0, al, 0.2 * al)
            plsc.addupdate_scatter(acc, [r16], jnp.exp(al))
        plsc.parallel_loop(0, FIRE, unroll=4)(edge)

    def scan_step(ci, cnt):
        base = ci * SCHUNK
        pltpu.sync_copy(src_hbm.at[pl.ds(base, SCHUNK)], srcb)
        pltpu.sync_copy(dst_hbm.at[pl.ds(base, SCHUNK)], dstb)

        def group(gi, cnt):
            sv = srcb[pl.ds(gi * 16, 16)]
            dv = dstb[pl.ds(gi * 16, 16)]
            m = (dv >= lo) & (dv < lo + R_A)
            cs = plsc.cumsum(jnp.where(m, 1, 0))
            pos = cnt + cs - 1
            plsc.store_scatter(pend_src, [pos], sv, mask=m)
            plsc.store_scatter(pend_row, [pos], dv - lo, mask=m)
            return cnt + cs[15]

        cnt = plsc.parallel_loop(0, SCHUNK // 16, unroll=4, carry=cnt)(group)
        n_full = cnt // FIRE

        def fire_j(j, carry):
            do_fire(j * FIRE)
            return carry
        lax.fori_loop(0, n_full, fire_j, 0)

        ro = n_full * FIRE
        for t in range(8):
            pend_src[pl.ds(t * 16, 16)] = pend_src[pl.ds(ro + t * 16, 16)]
            pend_row[pl.ds(t * 16, 16)] = pend_row[pl.ds(ro + t * 16, 16)]
        return cnt - ro

    cnt = lax.fori_loop(0, E // SCHUNK, scan_step, jnp.int32(0))

    for t in range(8):
        pend_src[pl.ds(cnt + t * 16, 16)] = jnp.zeros((16,), jnp.int32)
        pend_row[pl.ds(cnt + t * 16, 16)] = jnp.full((16,), R_A, jnp.int32)
    do_fire(0)

    pltpu.sync_copy(acc.at[pl.ds(0, R_A * 16)], out_hbm.at[pl.ds(lo * 16, R_A * 16)])


@functools.partial(
    pl.kernel,
    out_type=jax.ShapeDtypeStruct((E, 16), jnp.float32),
    scratch_types=[
        pltpu.VMEM((ECHUNK,), jnp.int32),
        pltpu.VMEM((ECHUNK,), jnp.int32),
        pltpu.VMEM((ECHUNK, 16), jnp.float32),
        pltpu.VMEM((ECHUNK, 16), jnp.float32),
        pltpu.VMEM((ECHUNK, 16), jnp.float32),
        pltpu.VMEM((ECHUNK, 16), jnp.float32),
        pltpu.SemaphoreType.DMA,
        pltpu.SemaphoreType.DMA,
        pltpu.SemaphoreType.DMA,
    ],
    **_SC_MESH,
)
def _attn_kernel(asrc_hbm, adst_hbm, den_hbm, src_hbm, dst_hbm, out_hbm,
                 srcc, dstc, gs, gd, gn, outb, sem1, sem2, sem3):
    e0 = _wid() * (E // NW)

    def chunk(ki, carry):
        base = e0 + ki * ECHUNK
        pltpu.sync_copy(src_hbm.at[pl.ds(base, ECHUNK)], srcc)
        pltpu.sync_copy(dst_hbm.at[pl.ds(base, ECHUNK)], dstc)
        ca = pltpu.async_copy(asrc_hbm.at[srcc], gs, sem1)
        cb = pltpu.async_copy(adst_hbm.at[dstc], gd, sem2)
        cc = pltpu.async_copy(den_hbm.at[dstc], gn, sem3)
        ca.wait()
        cb.wait()
        cc.wait()

        def edge(j):
            a_s = gs[j, pl.ds(0, 16)]
            a_d = gd[j, pl.ds(0, 16)]
            dn = gn[j, pl.ds(0, 16)]
            al = a_s + a_d
            al = jnp.where(al >= 0, al, 0.2 * al)
            outb[j, pl.ds(0, 16)] = jnp.exp(al) / (dn + 1e-16)
        plsc.parallel_loop(0, ECHUNK, unroll=4)(edge)
        pltpu.sync_copy(outb, out_hbm.at[pl.ds(base, ECHUNK)])
        return carry

    lax.fori_loop(0, (E // NW) // ECHUNK, chunk, 0)


@functools.partial(
    pl.kernel,
    out_type=jax.ShapeDtypeStruct((PAD_N * H * D,), jnp.float32),
    scratch_types=[
        pltpu.VMEM((SCHUNK_C,), jnp.int32),
        pltpu.VMEM((SCHUNK_C,), jnp.int32),
        pltpu.VMEM((SCHUNK_C + 160,), jnp.int32),
        pltpu.VMEM((SCHUNK_C + 160,), jnp.int32),
        pltpu.VMEM((SCHUNK_C + 160,), jnp.int32),
        pltpu.VMEM((FIRE,), jnp.int32),
        pltpu.VMEM((FIRE,), jnp.int32),
        pltpu.VMEM((FIRE, D), jnp.float32),
        pltpu.VMEM((FIRE, 16), jnp.float32),
        pltpu.VMEM(((R_C + 4) * H * D,), jnp.float32),
        pltpu.SemaphoreType.DMA,
        pltpu.SemaphoreType.DMA,
    ],
    **_SC_MESH,
)
def _gatagg_kernel(x_hbm, attn_hbm, src_hbm, dst_hbm, out_hbm,
                   srcb, dstb, pend_src, pend_row, pend_id,
                   fire_idx, fire_id, xbuf, abuf, acc, sem1, sem2):
    w = _wid()
    iota = lax.iota(jnp.int32, 16)
    HD = H * D

    def pass_body(p, pcarry):
        lo = (p * NW + w) * R_C
        _zero_flat(acc, (R_C + 4) * HD)

        def do_fire(off):
            for t in range(FIRE // 16):
                fire_idx[pl.ds(t * 16, 16)] = pend_src[pl.ds(off + t * 16, 16)]
                fire_id[pl.ds(t * 16, 16)] = pend_id[pl.ds(off + t * 16, 16)]
            ca = pltpu.async_copy(x_hbm.at[fire_idx], xbuf, sem1)
            cb = pltpu.async_copy(attn_hbm.at[fire_id], abuf, sem2)
            ca.wait()
            cb.wait()

            def edge(j):
                rowv = plsc.load_gather(pend_row, [_splat(off + j)])
                rbase = rowv * HD + iota
                jv = _splat(j)
                coefs = [plsc.load_gather(abuf, [jv, _splat(h)]) for h in range(H)]
                for k in range(8):
                    vals = xbuf[j, pl.ds(k * 16, 16)]
                    for h in range(H):
                        plsc.addupdate_scatter(
                            acc, [rbase + (h * D + k * 16)], vals * coefs[h])
            plsc.parallel_loop(0, FIRE, unroll=2)(edge)

        def scan_step(ci, cnt):
            base = ci * SCHUNK_C
            pltpu.sync_copy(src_hbm.at[pl.ds(base, SCHUNK_C)], srcb)
            pltpu.sync_copy(dst_hbm.at[pl.ds(base, SCHUNK_C)], dstb)

            def group(gi, cnt):
                sv = srcb[pl.ds(gi * 16, 16)]
                dv = dstb[pl.ds(gi * 16, 16)]
                ev = base + gi * 16 + iota
                m = (dv >= lo) & (dv < lo + R_C)
                cs = plsc.cumsum(jnp.where(m, 1, 0))
                pos = cnt + cs - 1
                plsc.store_scatter(pend_src, [pos], sv, mask=m)
                plsc.store_scatter(pend_row, [pos], dv - lo, mask=m)
                plsc.store_scatter(pend_id, [pos], ev, mask=m)
                return cnt + cs[15]

            cnt = plsc.parallel_loop(0, SCHUNK_C // 16, unroll=4, carry=cnt)(group)
            n_full = cnt // FIRE

            def fire_j(j, carry):
                do_fire(j * FIRE)
                return carry
            lax.fori_loop(0, n_full, fire_j, 0)

            ro = n_full * FIRE
            for t in range(8):
                pend_src[pl.ds(t * 16, 16)] = pend_src[pl.ds(ro + t * 16, 16)]
                pend_row[pl.ds(t * 16, 16)] = pend_row[pl.ds(ro + t * 16, 16)]
                pend_id[pl.ds(t * 16, 16)] = pend_id[pl.ds(ro + t * 16, 16)]
            return cnt - ro

        cnt = lax.fori_loop(0, E // SCHUNK_C, scan_step, jnp.int32(0))

        for t in range(8):
            pend_src[pl.ds(cnt + t * 16, 16)] = jnp.zeros((16,), jnp.int32)
            pend_row[pl.ds(cnt + t * 16, 16)] = jnp.full((16,), R_C, jnp.int32)
            pend_id[pl.ds(cnt + t * 16, 16)] = jnp.zeros((16,), jnp.int32)
        do_fire(0)

        pltpu.sync_copy(acc.at[pl.ds(0, R_C * HD)],
                        out_hbm.at[pl.ds(lo * HD, R_C * HD)])
        return pcarry

    lax.fori_loop(0, NPASS, pass_body, 0)


def _proj_body(x_ref, w_ref, att_ref, out_ref):
    w3 = w_ref[...].reshape(D, H, D)
    v = jnp.sum(w3 * att_ref[...][None, :, :], axis=-1)
    vpad = jnp.concatenate([v, jnp.zeros((D, 16 - H), jnp.float32)], axis=1)
    out_ref[...] = jnp.dot(x_ref[...], vpad, preferred_element_type=jnp.float32)


def _proj(x, W, att, blk):
    n = x.shape[0]
    return pl.pallas_call(
        _proj_body,
        grid=(n // blk,),
        in_specs=[
            pl.BlockSpec((blk, D), lambda i: (i, 0)),
            pl.BlockSpec((D, H * D), lambda i: (0, 0)),
            pl.BlockSpec((H, D), lambda i: (0, 0)),
        ],
        out_specs=pl.BlockSpec((blk, 16), lambda i: (i, 0)),
        out_shape=jax.ShapeDtypeStruct((n, 16), jnp.float32),
    )(x, W, att)


def _job_body(agg_ref, xj_ref, wrel_ref, wroot_ref, b_ref, out_ref):
    r = (jnp.dot(agg_ref[...], wrel_ref[...], preferred_element_type=jnp.float32)
         + jnp.dot(xj_ref[...], wroot_ref[...], preferred_element_type=jnp.float32)
         + b_ref[...])
    out_ref[...] = _leaky(r)


def _job_out(agg, x_job, W_rel, W_root, b_rel):
    blk = 2000
    return pl.pallas_call(
        _job_body,
        grid=(N_JOB // blk,),
        in_specs=[
            pl.BlockSpec((blk, D), lambda i: (i, 0)),
            pl.BlockSpec((blk, D), lambda i: (i, 0)),
            pl.BlockSpec((D, D), lambda i: (0, 0)),
            pl.BlockSpec((D, D), lambda i: (0, 0)),
            pl.BlockSpec((D,), lambda i: (0,)),
        ],
        out_specs=pl.BlockSpec((blk, D), lambda i: (i, 0)),
        out_shape=jax.ShapeDtypeStruct((N_JOB, D), jnp.float32),
    )(agg, x_job, W_rel, W_root, b_rel)


def _gatout_body(c_ref, ws_ref, b_ref, out_ref):
    r = jnp.dot(c_ref[...], ws_ref[...], preferred_element_type=jnp.float32)
    out_ref[...] = _leaky(r * (1.0 / H) + b_ref[...])


def _gat_out(C, W_stack, b_gat):
    blk = 2000
    return pl.pallas_call(
        _gatout_body,
        grid=(N_RES // blk,),
        in_specs=[
            pl.BlockSpec((blk, H * D), lambda i: (i, 0)),
            pl.BlockSpec((H * D, D), lambda i: (0, 0)),
            pl.BlockSpec((D,), lambda i: (0,)),
        ],
        out_specs=pl.BlockSpec((blk, D), lambda i: (i, 0)),
        out_shape=jax.ShapeDtypeStruct((N_RES, D), jnp.float32),
    )(C, W_stack, b_gat)


def _mlp_body(comb_ref, w1_ref, b1_ref, w2_ref, b2_ref, w3_ref, b3_ref, out_ref):
    comb = comb_ref[...]
    h1 = _leaky(jnp.dot(comb, w1_ref[...], preferred_element_type=jnp.float32) + b1_ref[...])
    h2 = _leaky(jnp.dot(h1, w2_ref[...], preferred_element_type=jnp.float32) + b2_ref[...])
    score = jnp.sum(h2 * w3_ref[...].reshape(1, D), axis=1) + b3_ref[0]
    out_ref[...] = score


def _mlp(comb, W1, b1, W2, b2, W3, b3):
    n = comb.shape[0]
    return pl.pallas_call(
        _mlp_body,
        out_shape=jax.ShapeDtypeStruct((n,), jnp.float32),
    )(comb, W1, b1, W2, b2, W3, b3)


def kernel(x_skill, x_job, x_resume, edge_index_sj, edge_index_sr, W_rel, b_rel, W_root, W_src, W_dst, att_src, att_dst, b_gat, W1, b1, W2, b2, W3, b3):
    src_j = edge_index_sj[0]
    dst_j = edge_index_sj[1]
    src_r = edge_index_sr[0]
    dst_r = edge_index_sr[1]

    agg = _gc_kernel(x_skill, src_j, dst_j).reshape(PAD_N, D)[:N_JOB]
    xj = _job_out(agg, x_job, W_rel, W_root, b_rel)

    a_src = _proj(x_skill, W_src, att_src, 2000)
    a_dst = _proj(x_resume, W_dst, att_dst, 2000)
    a_dst_pad = jnp.concatenate(
        [a_dst, jnp.zeros((PAD_N - N_RES, 16), jnp.float32)], axis=0)

    den = _den_kernel(a_src, a_dst_pad.reshape(-1), src_r, dst_r).reshape(PAD_N, 16)
    attn_pad = _attn_kernel(a_src, a_dst_pad, den, src_r, dst_r)
    attn = attn_pad[:, :H]

    C = _gatagg_kernel(x_skill, attn_pad, src_r, dst_r).reshape(PAD_N, H * D)[:N_RES]
    W_stack = W_src.reshape(D, H, D).transpose(1, 0, 2).reshape(H * D, D)
    xr = _gat_out(C, W_stack, b_gat)

    comb = jnp.concatenate([xj, xr], axis=1)
    score = _mlp(comb, W1, b1, W2, b2, W3, b3)
    return (score, attn)


# speedup vs baseline: 12.8024x; 1.1061x over previous
import functools

import jax
import jax.numpy as jnp
from jax import lax
from jax.experimental import pallas as pl
from jax.experimental.pallas import tpu as pltpu
from jax.experimental.pallas import tpu_sc as plsc

N_SKILL = 100000
N_JOB = 10000
N_RES = 10000
D = 128
H = 8
E = 320000

NW = 32
R_A = 320
PAD_N = R_A * NW
FIRE = 128
SCHUNK = 8000
R_C = 80
NPASS = 4
SCHUNK_C = 4000
ECHUNK = 80

_SC_MESH = dict(
    mesh=plsc.VectorSubcoreMesh(core_axis_name="c", subcore_axis_name="s"),
    compiler_params=pltpu.CompilerParams(
        needs_layout_passes=False, use_tc_tiling_on_sc=False),
)


def _leaky(x, s=0.01):
    return jnp.where(x >= 0, x, s * x)


def _wid():
    return lax.axis_index("s") * 2 + lax.axis_index("c")


def _zero_flat(ref, n):
    def _z(i, carry):
        for kk in range(8):
            ref[pl.ds(i * 128 + kk * 16, 16)] = jnp.zeros((16,), jnp.float32)
        return carry
    lax.fori_loop(0, n // 128, _z, 0)


def _splat(x):
    return jnp.full((16,), x, jnp.int32)


@functools.partial(
    pl.kernel,
    out_type=jax.ShapeDtypeStruct((PAD_N * D,), jnp.float32),
    scratch_types=[
        pltpu.VMEM((SCHUNK,), jnp.int32),
        pltpu.VMEM((SCHUNK,), jnp.int32),
        pltpu.VMEM((SCHUNK + 160,), jnp.int32),
        pltpu.VMEM((SCHUNK + 160,), jnp.int32),
        pltpu.VMEM((FIRE,), jnp.int32),
        pltpu.VMEM((FIRE, D), jnp.float32),
        pltpu.VMEM(((R_A + 8) * D,), jnp.float32),
        pltpu.SemaphoreType.DMA,
    ],
    **_SC_MESH,
)
def _gc_kernel(x_hbm, src_hbm, dst_hbm, out_hbm,
               srcb, dstb, pend_src, pend_row, fire_idx, xbuf, acc, sem):
    lo = _wid() * R_A
    iota = lax.iota(jnp.int32, 16)
    _zero_flat(acc, (R_A + 8) * D)

    def do_fire_full(off):
        for t in range(FIRE // 16):
            fire_idx[pl.ds(t * 16, 16)] = pend_src[pl.ds(off + t * 16, 16)]
        pltpu.async_copy(x_hbm.at[fire_idx], xbuf, sem).wait()

        def edge(j):
            rowv = plsc.load_gather(pend_row, [_splat(off + j)])
            rbase = rowv * D + iota
            for k in range(8):
                vals = xbuf[j, pl.ds(k * 16, 16)]
                plsc.addupdate_scatter(acc, [rbase + (k * 16)], vals)
        plsc.parallel_loop(0, FIRE, unroll=2)(edge)

    def scan_step(ci, cnt):
        base = ci * SCHUNK
        pltpu.sync_copy(src_hbm.at[pl.ds(base, SCHUNK)], srcb)
        pltpu.sync_copy(dst_hbm.at[pl.ds(base, SCHUNK)], dstb)

        def group(gi, cnt):
            sv = srcb[pl.ds(gi * 16, 16)]
            dv = dstb[pl.ds(gi * 16, 16)]
            m = (dv >= lo) & (dv < lo + R_A)
            cs = plsc.cumsum(jnp.where(m, 1, 0))
            pos = cnt + cs - 1
            plsc.store_scatter(pend_src, [pos], sv, mask=m)
            plsc.store_scatter(pend_row, [pos], dv - lo, mask=m)
            return cnt + cs[15]

        cnt = plsc.parallel_loop(0, SCHUNK // 16, unroll=4, carry=cnt)(group)
        n_full = cnt // FIRE

        def fire_j(j, carry):
            do_fire_full(j * FIRE)
            return carry
        lax.fori_loop(0, n_full, fire_j, 0)

        ro = n_full * FIRE
        for t in range(8):
            pend_src[pl.ds(t * 16, 16)] = pend_src[pl.ds(ro + t * 16, 16)]
            pend_row[pl.ds(t * 16, 16)] = pend_row[pl.ds(ro + t * 16, 16)]
        return cnt - ro

    cnt = lax.fori_loop(0, E // SCHUNK, scan_step, jnp.int32(0))

    for t in range(8):
        pend_src[pl.ds(cnt + t * 16, 16)] = jnp.zeros((16,), jnp.int32)
        pend_row[pl.ds(cnt + t * 16, 16)] = jnp.full((16,), R_A, jnp.int32)
    do_fire_full(0)

    pltpu.sync_copy(acc.at[pl.ds(0, R_A * D)], out_hbm.at[pl.ds(lo * D, R_A * D)])


@functools.partial(
    pl.kernel,
    out_type=jax.ShapeDtypeStruct((PAD_N * 16,), jnp.float32),
    scratch_types=[
        pltpu.VMEM((SCHUNK,), jnp.int32),
        pltpu.VMEM((SCHUNK,), jnp.int32),
        pltpu.VMEM((SCHUNK + 160,), jnp.int32),
        pltpu.VMEM((SCHUNK + 160,), jnp.int32),
        pltpu.VMEM((FIRE,), jnp.int32),
        pltpu.VMEM((FIRE, 16), jnp.float32),
        pltpu.VMEM((R_A * 16,), jnp.float32),
        pltpu.VMEM(((R_A + 8) * 16,), jnp.float32),
        pltpu.SemaphoreType.DMA,
    ],
    **_SC_MESH,
)
def _den_kernel(asrc_hbm, adstf_hbm, src_hbm, dst_hbm, out_hbm,
                srcb, dstb, pend_src, pend_row, fire_idx, abuf, adst_own, acc, sem):
    lo = _wid() * R_A
    iota = lax.iota(jnp.int32, 16)
    _zero_flat(acc, (R_A + 8) * 16)
    pltpu.sync_copy(adstf_hbm.at[pl.ds(lo * 16, R_A * 16)], adst_own)

    def do_fire(off):
        for t in range(FIRE // 16):
            fire_idx[pl.ds(t * 16, 16)] = pend_src[pl.ds(off + t * 16, 16)]
        pltpu.async_copy(asrc_hbm.at[fire_idx], abuf, sem).wait()

        def edge(j):
            rowv = plsc.load_gather(pend_row, [_splat(off + j)])
            r16 = rowv * 16 + iota
            a_s = abuf[j, pl.ds(0, 16)]
            a_d = plsc.load_gather(adst_own, [r16])
            al = a_s + a_d
            al = jnp.where(al >= 0, al, 0.2 * al)
            plsc.addupdate_scatter(acc, [r16], jnp.exp(al))
        plsc.parallel_loop(0, FIRE, unroll=4)(edge)

    def scan_step(ci, cnt):
        base = ci * SCHUNK
        pltpu.sync_copy(src_hbm.at[pl.ds(base, SCHUNK)], srcb)
        pltpu.sync_copy(dst_hbm.at[pl.ds(base, SCHUNK)], dstb)

        def group(gi, cnt):
            sv = srcb[pl.ds(gi * 16, 16)]
            dv = dstb[pl.ds(gi * 16, 16)]
            m = (dv >= lo) & (dv < lo + R_A)
            cs = plsc.cumsum(jnp.where(m, 1, 0))
            pos = cnt + cs - 1
            plsc.store_scatter(pend_src, [pos], sv, mask=m)
            plsc.store_scatter(pend_row, [pos], dv - lo, mask=m)
            return cnt + cs[15]

        cnt = plsc.parallel_loop(0, SCHUNK // 16, unroll=4, carry=cnt)(group)
        n_full = cnt // FIRE

        def fire_j(j, carry):
            do_fire(j * FIRE)
            return carry
        lax.fori_loop(0, n_full, fire_j, 0)

        ro = n_full * FIRE
        for t in range(8):
            pend_src[pl.ds(t * 16, 16)] = pend_src[pl.ds(ro + t * 16, 16)]
            pend_row[pl.ds(t * 16, 16)] = pend_row[pl.ds(ro + t * 16, 16)]
        return cnt - ro

    cnt = lax.fori_loop(0, E // SCHUNK, scan_step, jnp.int32(0))

    for t in range(8):
        pend_src[pl.ds(cnt + t * 16, 16)] = jnp.zeros((16,), jnp.int32)
        pend_row[pl.ds(cnt + t * 16, 16)] = jnp.full((16,), R_A, jnp.int32)
    do_fire(0)

    pltpu.sync_copy(acc.at[pl.ds(0, R_A * 16)], out_hbm.at[pl.ds(lo * 16, R_A * 16)])


@functools.partial(
    pl.kernel,
    out_type=jax.ShapeDtypeStruct((E, 16), jnp.float32),
    scratch_types=[
        pltpu.VMEM((ECHUNK,), jnp.int32),
        pltpu.VMEM((ECHUNK,), jnp.int32),
        pltpu.VMEM((ECHUNK, 16), jnp.float32),
        pltpu.VMEM((ECHUNK, 16), jnp.float32),
        pltpu.VMEM((ECHUNK, 16), jnp.float32),
        pltpu.VMEM((ECHUNK, 16), jnp.float32),
        pltpu.SemaphoreType.DMA,
        pltpu.SemaphoreType.DMA,
        pltpu.SemaphoreType.DMA,
    ],
    **_SC_MESH,
)
def _attn_kernel(asrc_hbm, adst_hbm, den_hbm, src_hbm, dst_hbm, out_hbm,
                 srcc, dstc, gs, gd, gn, outb, sem1, sem2, sem3):
    e0 = _wid() * (E // NW)

    def chunk(ki, carry):
        base = e0 + ki * ECHUNK
        pltpu.sync_copy(src_hbm.at[pl.ds(base, ECHUNK)], srcc)
        pltpu.sync_copy(dst_hbm.at[pl.ds(base, ECHUNK)], dstc)
        ca = pltpu.async_copy(asrc_hbm.at[srcc], gs, sem1)
        cb = pltpu.async_copy(adst_hbm.at[dstc], gd, sem2)
        cc = pltpu.async_copy(den_hbm.at[dstc], gn, sem3)
        ca.wait()
        cb.wait()
        cc.wait()

        def edge(j):
            a_s = gs[j, pl.ds(0, 16)]
            a_d = gd[j, pl.ds(0, 16)]
            dn = gn[j, pl.ds(0, 16)]
            al = a_s + a_d
            al = jnp.where(al >= 0, al, 0.2 * al)
            outb[j, pl.ds(0, 16)] = jnp.exp(al) / (dn + 1e-16)
        plsc.parallel_loop(0, ECHUNK, unroll=4)(edge)
        pltpu.sync_copy(outb, out_hbm.at[pl.ds(base, ECHUNK)])
        return carry

    lax.fori_loop(0, (E // NW) // ECHUNK, chunk, 0)


@functools.partial(
    pl.kernel,
    out_type=jax.ShapeDtypeStruct((PAD_N * H * D,), jnp.float32),
    scratch_types=[
        pltpu.VMEM((SCHUNK_C,), jnp.int32),
        pltpu.VMEM((SCHUNK_C,), jnp.int32),
        pltpu.VMEM((SCHUNK_C + 160,), jnp.int32),
        pltpu.VMEM((SCHUNK_C + 160,), jnp.int32),
        pltpu.VMEM((SCHUNK_C + 160,), jnp.int32),
        pltpu.VMEM((FIRE,), jnp.int32),
        pltpu.VMEM((FIRE,), jnp.int32),
        pltpu.VMEM((FIRE, D), jnp.float32),
        pltpu.VMEM((FIRE, 16), jnp.float32),
        pltpu.VMEM(((R_C + 1) * H * D,), jnp.float32),
        pltpu.SemaphoreType.DMA,
        pltpu.SemaphoreType.DMA,
    ],
    **_SC_MESH,
)
def _gatagg_kernel(x_hbm, attn_hbm, src_hbm, dst_hbm, out_hbm,
                   srcb, dstb, pend_src, pend_row, pend_id,
                   fire_idx, fire_id, xbuf, abuf, acc, sem1, sem2):
    w = _wid()
    iota = lax.iota(jnp.int32, 16)
    HD = H * D

    def pass_body(p, pcarry):
        lo = (p * NW + w) * R_C
        _zero_flat(acc, (R_C + 1) * HD)

        def do_fire(off):
            for t in range(FIRE // 16):
                fire_idx[pl.ds(t * 16, 16)] = pend_src[pl.ds(off + t * 16, 16)]
                fire_id[pl.ds(t * 16, 16)] = pend_id[pl.ds(off + t * 16, 16)]
            ca = pltpu.async_copy(x_hbm.at[fire_idx], xbuf, sem1)
            cb = pltpu.async_copy(attn_hbm.at[fire_id], abuf, sem2)
            ca.wait()
            cb.wait()

            def edge(j):
                rowv = plsc.load_gather(pend_row, [_splat(off + j)])
                rbase = rowv * HD + iota
                jv = _splat(j)
                coefs = [plsc.load_gather(abuf, [jv, _splat(h)]) for h in range(H)]
                for k in range(8):
                    vals = xbuf[j, pl.ds(k * 16, 16)]
                    for h in range(H):
                        plsc.addupdate_scatter(
                            acc, [rbase + (h * D + k * 16)], vals * coefs[h])
            plsc.parallel_loop(0, FIRE, unroll=2)(edge)

        def scan_step(ci, cnt):
            base = ci * SCHUNK_C
            pltpu.sync_copy(src_hbm.at[pl.ds(base, SCHUNK_C)], srcb)
            pltpu.sync_copy(dst_hbm.at[pl.ds(base, SCHUNK_C)], dstb)

            def group(gi, cnt):
                sv = srcb[pl.ds(gi * 16, 16)]
                dv = dstb[pl.ds(gi * 16, 16)]
                ev = base + gi * 16 + iota
                m = (dv >= lo) & (dv < lo + R_C)
                cs = plsc.cumsum(jnp.where(m, 1, 0))
                pos = cnt + cs - 1
                plsc.store_scatter(pend_src, [pos], sv, mask=m)
                plsc.store_scatter(pend_row, [pos], dv - lo, mask=m)
                plsc.store_scatter(pend_id, [pos], ev, mask=m)
                return cnt + cs[15]

            cnt = plsc.parallel_loop(0, SCHUNK_C // 16, unroll=4, carry=cnt)(group)
            n_full = cnt // FIRE

            def fire_j(j, carry):
                do_fire(j * FIRE)
                return carry
            lax.fori_loop(0, n_full, fire_j, 0)

            ro = n_full * FIRE
            for t in range(8):
                pend_src[pl.ds(t * 16, 16)] = pend_src[pl.ds(ro + t * 16, 16)]
                pend_row[pl.ds(t * 16, 16)] = pend_row[pl.ds(ro + t * 16, 16)]
                pend_id[pl.ds(t * 16, 16)] = pend_id[pl.ds(ro + t * 16, 16)]
            return cnt - ro

        cnt = lax.fori_loop(0, E // SCHUNK_C, scan_step, jnp.int32(0))

        for t in range(8):
            pend_src[pl.ds(cnt + t * 16, 16)] = jnp.zeros((16,), jnp.int32)
            pend_row[pl.ds(cnt + t * 16, 16)] = jnp.full((16,), R_C, jnp.int32)
            pend_id[pl.ds(cnt + t * 16, 16)] = jnp.zeros((16,), jnp.int32)
        do_fire(0)

        pltpu.sync_copy(acc.at[pl.ds(0, R_C * HD)],
                        out_hbm.at[pl.ds(lo * HD, R_C * HD)])
        return pcarry

    lax.fori_loop(0, NPASS, pass_body, 0)


def _proj_body(x_ref, w_ref, att_ref, out_ref):
    w3 = w_ref[...].reshape(D, H, D)
    v = jnp.sum(w3 * att_ref[...][None, :, :], axis=-1)
    vpad = jnp.concatenate([v, jnp.zeros((D, 16 - H), jnp.float32)], axis=1)
    out_ref[...] = jnp.dot(x_ref[...], vpad, preferred_element_type=jnp.float32)


def _proj(x, W, att, blk):
    n = x.shape[0]
    return pl.pallas_call(
        _proj_body,
        grid=(n // blk,),
        in_specs=[
            pl.BlockSpec((blk, D), lambda i: (i, 0)),
            pl.BlockSpec((D, H * D), lambda i: (0, 0)),
            pl.BlockSpec((H, D), lambda i: (0, 0)),
        ],
        out_specs=pl.BlockSpec((blk, 16), lambda i: (i, 0)),
        out_shape=jax.ShapeDtypeStruct((n, 16), jnp.float32),
    )(x, W, att)


def _job_body(agg_ref, xj_ref, wrel_ref, wroot_ref, b_ref, out_ref):
    r = (jnp.dot(agg_ref[...], wrel_ref[...], preferred_element_type=jnp.float32)
         + jnp.dot(xj_ref[...], wroot_ref[...], preferred_element_type=jnp.float32)
         + b_ref[...])
    out_ref[...] = _leaky(r)


def _job_out(agg, x_job, W_rel, W_root, b_rel):
    blk = 2000
    return pl.pallas_call(
        _job_body,
        grid=(N_JOB // blk,),
        in_specs=[
            pl.BlockSpec((blk, D), lambda i: (i, 0)),
            pl.BlockSpec((blk, D), lambda i: (i, 0)),
            pl.BlockSpec((D, D), lambda i: (0, 0)),
            pl.BlockSpec((D, D), lambda i: (0, 0)),
            pl.BlockSpec((D,), lambda i: (0,)),
        ],
        out_specs=pl.BlockSpec((blk, D), lambda i: (i, 0)),
        out_shape=jax.ShapeDtypeStruct((N_JOB, D), jnp.float32),
    )(agg, x_job, W_rel, W_root, b_rel)


def _gatout_body(c_ref, ws_ref, b_ref, out_ref):
    r = jnp.dot(c_ref[...], ws_ref[...], preferred_element_type=jnp.float32)
    out_ref[...] = _leaky(r * (1.0 / H) + b_ref[...])


def _gat_out(C, W_stack, b_gat):
    blk = 2000
    return pl.pallas_call(
        _gatout_body,
        grid=(N_RES // blk,),
        in_specs=[
            pl.BlockSpec((blk, H * D), lambda i: (i, 0)),
            pl.BlockSpec((H * D, D), lambda i: (0, 0)),
            pl.BlockSpec((D,), lambda i: (0,)),
        ],
        out_specs=pl.BlockSpec((blk, D), lambda i: (i, 0)),
        out_shape=jax.ShapeDtypeStruct((N_RES, D), jnp.float32),
    )(C, W_stack, b_gat)


def _mlp_body(comb_ref, w1_ref, b1_ref, w2_ref, b2_ref, w3_ref, b3_ref, out_ref):
    comb = comb_ref[...]
    h1 = _leaky(jnp.dot(comb, w1_ref[...], preferred_element_type=jnp.float32) + b1_ref[...])
    h2 = _leaky(jnp.dot(h1, w2_ref[...], preferred_element_type=jnp.float32) + b2_ref[...])
    score = jnp.sum(h2 * w3_ref[...].reshape(1, D), axis=1) + b3_ref[0]
    out_ref[...] = score


def _mlp(comb, W1, b1, W2, b2, W3, b3):
    n = comb.shape[0]
    return pl.pallas_call(
        _mlp_body,
        out_shape=jax.ShapeDtypeStruct((n,), jnp.float32),
    )(comb, W1, b1, W2, b2, W3, b3)


def kernel(x_skill, x_job, x_resume, edge_index_sj, edge_index_sr, W_rel, b_rel, W_root, W_src, W_dst, att_src, att_dst, b_gat, W1, b1, W2, b2, W3, b3):
    src_j = edge_index_sj[0]
    dst_j = edge_index_sj[1]
    src_r = edge_index_sr[0]
    dst_r = edge_index_sr[1]

    agg = _gc_kernel(x_skill, src_j, dst_j).reshape(PAD_N, D)[:N_JOB]
    xj = _job_out(agg, x_job, W_rel, W_root, b_rel)

    a_src = _proj(x_skill, W_src, att_src, 2000)
    a_dst = _proj(x_resume, W_dst, att_dst, 2000)
    a_dst_pad = jnp.concatenate(
        [a_dst, jnp.zeros((PAD_N - N_RES, 16), jnp.float32)], axis=0)

    den = _den_kernel(a_src, a_dst_pad.reshape(-1), src_r, dst_r).reshape(PAD_N, 16)
    attn_pad = _attn_kernel(a_src, a_dst_pad, den, src_r, dst_r)
    attn = attn_pad[:, :H]

    C = _gatagg_kernel(x_skill, attn_pad, src_r, dst_r).reshape(PAD_N, H * D)[:N_RES]
    W_stack = W_src.reshape(D, H, D).transpose(1, 0, 2).reshape(H * D, D)
    xr = _gat_out(C, W_stack, b_gat)

    comb = jnp.concatenate([xj, xr], axis=1)
    score = _mlp(comb, W1, b1, W2, b2, W3, b3)
    return (score, attn)


# speedup vs baseline: 14.1684x; 1.1067x over previous
import functools

import jax
import jax.numpy as jnp
from jax import lax
from jax.experimental import pallas as pl
from jax.experimental.pallas import tpu as pltpu
from jax.experimental.pallas import tpu_sc as plsc

N_SKILL = 100000
N_JOB = 10000
N_RES = 10000
D = 128
H = 8
E = 320000

NW = 32
R_A = 320
PAD_N = R_A * NW
FIRE = 128
SCHUNK = 8000
R_C = 80
NPASS = 4
SCHUNK_C = 4000
ECHUNK = 80

_SC_MESH = dict(
    mesh=plsc.VectorSubcoreMesh(core_axis_name="c", subcore_axis_name="s"),
    compiler_params=pltpu.CompilerParams(
        needs_layout_passes=False, use_tc_tiling_on_sc=False),
)


def _leaky(x, s=0.01):
    return jnp.where(x >= 0, x, s * x)


def _wid():
    return lax.axis_index("s") * 2 + lax.axis_index("c")


def _zero_flat(ref, n):
    def _z(i, carry):
        for kk in range(8):
            ref[pl.ds(i * 128 + kk * 16, 16)] = jnp.zeros((16,), jnp.float32)
        return carry
    lax.fori_loop(0, n // 128, _z, 0)


def _splat(x):
    return jnp.full((16,), x, jnp.int32)


@functools.partial(
    pl.kernel,
    out_type=jax.ShapeDtypeStruct((PAD_N * D,), jnp.float32),
    scratch_types=[
        pltpu.VMEM((SCHUNK,), jnp.int32),
        pltpu.VMEM((SCHUNK,), jnp.int32),
        pltpu.VMEM((SCHUNK + 160,), jnp.int32),
        pltpu.VMEM((SCHUNK + 160,), jnp.int32),
        pltpu.VMEM((FIRE,), jnp.int32),
        pltpu.VMEM((FIRE, D), jnp.float32),
        pltpu.VMEM(((R_A + 8) * D,), jnp.float32),
        pltpu.SemaphoreType.DMA,
        pltpu.SemaphoreType.DMA,
    ],
    **_SC_MESH,
)
def _gc_kernel(x_hbm, src_hbm, dst_hbm, out_hbm,
               srcb, dstb, pend_src, pend_row, fire_idx, xbuf, acc, sem, sem2):
    lo = _wid() * R_A
    iota = lax.iota(jnp.int32, 16)
    _zero_flat(acc, (R_A + 8) * D)

    def do_fire_full(off):
        for t in range(FIRE // 16):
            fire_idx[pl.ds(t * 16, 16)] = pend_src[pl.ds(off + t * 16, 16)]
        pltpu.async_copy(x_hbm.at[fire_idx], xbuf, sem).wait()

        def edge(j):
            rowv = plsc.load_gather(pend_row, [_splat(off + j)])
            rbase = rowv * D + iota
            for k in range(8):
                vals = xbuf[j, pl.ds(k * 16, 16)]
                plsc.addupdate_scatter(acc, [rbase + (k * 16)], vals)
        plsc.parallel_loop(0, FIRE, unroll=2)(edge)

    def scan_step(ci, cnt):
        base = ci * SCHUNK
        ca = pltpu.async_copy(src_hbm.at[pl.ds(base, SCHUNK)], srcb, sem)
        cb = pltpu.async_copy(dst_hbm.at[pl.ds(base, SCHUNK)], dstb, sem2)
        ca.wait()
        cb.wait()

        def group(gi, cnt):
            sv = srcb[pl.ds(gi * 16, 16)]
            dv = dstb[pl.ds(gi * 16, 16)]
            m = (dv >= lo) & (dv < lo + R_A)
            cs = plsc.cumsum(jnp.where(m, 1, 0))
            pos = cnt + cs - 1
            plsc.store_scatter(pend_src, [pos], sv, mask=m)
            plsc.store_scatter(pend_row, [pos], dv - lo, mask=m)
            return cnt + cs[15]

        cnt = plsc.parallel_loop(0, SCHUNK // 16, unroll=4, carry=cnt)(group)
        n_full = cnt // FIRE

        def fire_j(j, carry):
            do_fire_full(j * FIRE)
            return carry
        lax.fori_loop(0, n_full, fire_j, 0)

        ro = n_full * FIRE
        for t in range(8):
            pend_src[pl.ds(t * 16, 16)] = pend_src[pl.ds(ro + t * 16, 16)]
            pend_row[pl.ds(t * 16, 16)] = pend_row[pl.ds(ro + t * 16, 16)]
        return cnt - ro

    cnt = lax.fori_loop(0, E // SCHUNK, scan_step, jnp.int32(0))

    for t in range(8):
        pend_src[pl.ds(cnt + t * 16, 16)] = jnp.zeros((16,), jnp.int32)
        pend_row[pl.ds(cnt + t * 16, 16)] = jnp.full((16,), R_A, jnp.int32)
    do_fire_full(0)

    pltpu.sync_copy(acc.at[pl.ds(0, R_A * D)], out_hbm.at[pl.ds(lo * D, R_A * D)])


@functools.partial(
    pl.kernel,
    out_type=jax.ShapeDtypeStruct((PAD_N * 16,), jnp.float32),
    scratch_types=[
        pltpu.VMEM((SCHUNK,), jnp.int32),
        pltpu.VMEM((SCHUNK,), jnp.int32),
        pltpu.VMEM((SCHUNK + 160,), jnp.int32),
        pltpu.VMEM((SCHUNK + 160,), jnp.int32),
        pltpu.VMEM((FIRE,), jnp.int32),
        pltpu.VMEM((FIRE, 16), jnp.float32),
        pltpu.VMEM((R_A * 16,), jnp.float32),
        pltpu.VMEM(((R_A + 8) * 16,), jnp.float32),
        pltpu.SemaphoreType.DMA,
        pltpu.SemaphoreType.DMA,
    ],
    **_SC_MESH,
)
def _den_kernel(asrc_hbm, adstf_hbm, src_hbm, dst_hbm, out_hbm,
                srcb, dstb, pend_src, pend_row, fire_idx, abuf, adst_own, acc, sem, sem2):
    lo = _wid() * R_A
    iota = lax.iota(jnp.int32, 16)
    _zero_flat(acc, (R_A + 8) * 16)
    pltpu.sync_copy(adstf_hbm.at[pl.ds(lo * 16, R_A * 16)], adst_own)

    def do_fire(off):
        for t in range(FIRE // 16):
            fire_idx[pl.ds(t * 16, 16)] = pend_src[pl.ds(off + t * 16, 16)]
        pltpu.async_copy(asrc_hbm.at[fire_idx], abuf, sem).wait()

        def edge(j):
            rowv = plsc.load_gather(pend_row, [_splat(off + j)])
            r16 = rowv * 16 + iota
            a_s = abuf[j, pl.ds(0, 16)]
            a_d = plsc.load_gather(adst_own, [r16])
            al = a_s + a_d
            al = jnp.where(al >= 0, al, 0.2 * al)
            plsc.addupdate_scatter(acc, [r16], jnp.exp(al))
        plsc.parallel_loop(0, FIRE, unroll=4)(edge)

    def scan_step(ci, cnt):
        base = ci * SCHUNK
        ca = pltpu.async_copy(src_hbm.at[pl.ds(base, SCHUNK)], srcb, sem)
        cb = pltpu.async_copy(dst_hbm.at[pl.ds(base, SCHUNK)], dstb, sem2)
        ca.wait()
        cb.wait()

        def group(gi, cnt):
            sv = srcb[pl.ds(gi * 16, 16)]
            dv = dstb[pl.ds(gi * 16, 16)]
            m = (dv >= lo) & (dv < lo + R_A)
            cs = plsc.cumsum(jnp.where(m, 1, 0))
            pos = cnt + cs - 1
            plsc.store_scatter(pend_src, [pos], sv, mask=m)
            plsc.store_scatter(pend_row, [pos], dv - lo, mask=m)
            return cnt + cs[15]

        cnt = plsc.parallel_loop(0, SCHUNK // 16, unroll=4, carry=cnt)(group)
        n_full = cnt // FIRE

        def fire_j(j, carry):
            do_fire(j * FIRE)
            return carry
        lax.fori_loop(0, n_full, fire_j, 0)

        ro = n_full * FIRE
        for t in range(8):
            pend_src[pl.ds(t * 16, 16)] = pend_src[pl.ds(ro + t * 16, 16)]
            pend_row[pl.ds(t * 16, 16)] = pend_row[pl.ds(ro + t * 16, 16)]
        return cnt - ro

    cnt = lax.fori_loop(0, E // SCHUNK, scan_step, jnp.int32(0))

    for t in range(8):
        pend_src[pl.ds(cnt + t * 16, 16)] = jnp.zeros((16,), jnp.int32)
        pend_row[pl.ds(cnt + t * 16, 16)] = jnp.full((16,), R_A, jnp.int32)
    do_fire(0)

    pltpu.sync_copy(acc.at[pl.ds(0, R_A * 16)], out_hbm.at[pl.ds(lo * 16, R_A * 16)])


@functools.partial(
    pl.kernel,
    out_type=jax.ShapeDtypeStruct((E, 16), jnp.float32),
    scratch_types=[
        pltpu.VMEM((ECHUNK,), jnp.int32),
        pltpu.VMEM((ECHUNK,), jnp.int32),
        pltpu.VMEM((ECHUNK, 16), jnp.float32),
        pltpu.VMEM((ECHUNK, 16), jnp.float32),
        pltpu.VMEM((ECHUNK, 16), jnp.float32),
        pltpu.VMEM((ECHUNK, 16), jnp.float32),
        pltpu.SemaphoreType.DMA,
        pltpu.SemaphoreType.DMA,
        pltpu.SemaphoreType.DMA,
    ],
    **_SC_MESH,
)
def _attn_kernel(asrc_hbm, adst_hbm, den_hbm, src_hbm, dst_hbm, out_hbm,
                 srcc, dstc, gs, gd, gn, outb, sem1, sem2, sem3):
    e0 = _wid() * (E // NW)

    def chunk(ki, carry):
        base = e0 + ki * ECHUNK
        sa = pltpu.async_copy(src_hbm.at[pl.ds(base, ECHUNK)], srcc, sem1)
        sb = pltpu.async_copy(dst_hbm.at[pl.ds(base, ECHUNK)], dstc, sem2)
        sa.wait()
        sb.wait()
        ca = pltpu.async_copy(asrc_hbm.at[srcc], gs, sem1)
        cb = pltpu.async_copy(adst_hbm.at[dstc], gd, sem2)
        cc = pltpu.async_copy(den_hbm.at[dstc], gn, sem3)
        ca.wait()
        cb.wait()
        cc.wait()

        def edge(j):
            a_s = gs[j, pl.ds(0, 16)]
            a_d = gd[j, pl.ds(0, 16)]
            dn = gn[j, pl.ds(0, 16)]
            al = a_s + a_d
            al = jnp.where(al >= 0, al, 0.2 * al)
            outb[j, pl.ds(0, 16)] = jnp.exp(al) / (dn + 1e-16)
        plsc.parallel_loop(0, ECHUNK, unroll=4)(edge)
        pltpu.sync_copy(outb, out_hbm.at[pl.ds(base, ECHUNK)])
        return carry

    lax.fori_loop(0, (E // NW) // ECHUNK, chunk, 0)


@functools.partial(
    pl.kernel,
    out_type=jax.ShapeDtypeStruct((PAD_N * H * D,), jnp.float32),
    scratch_types=[
        pltpu.VMEM((SCHUNK_C,), jnp.int32),
        pltpu.VMEM((SCHUNK_C,), jnp.int32),
        pltpu.VMEM((SCHUNK_C + 160,), jnp.int32),
        pltpu.VMEM((SCHUNK_C + 160,), jnp.int32),
        pltpu.VMEM((SCHUNK_C + 160,), jnp.int32),
        pltpu.VMEM((FIRE,), jnp.int32),
        pltpu.VMEM((FIRE,), jnp.int32),
        pltpu.VMEM((FIRE, D), jnp.float32),
        pltpu.VMEM((FIRE, 16), jnp.float32),
        pltpu.VMEM(((R_C + 1) * H * D,), jnp.float32),
        pltpu.SemaphoreType.DMA,
        pltpu.SemaphoreType.DMA,
    ],
    **_SC_MESH,
)
def _gatagg_kernel(x_hbm, attn_hbm, src_hbm, dst_hbm, out_hbm,
                   srcb, dstb, pend_src, pend_row, pend_id,
                   fire_idx, fire_id, xbuf, abuf, acc, sem1, sem2):
    w = _wid()
    iota = lax.iota(jnp.int32, 16)
    HD = H * D

    def pass_body(p, pcarry):
        lo = (p * NW + w) * R_C
        _zero_flat(acc, (R_C + 1) * HD)

        def do_fire(off):
            for t in range(FIRE // 16):
                fire_idx[pl.ds(t * 16, 16)] = pend_src[pl.ds(off + t * 16, 16)]
                fire_id[pl.ds(t * 16, 16)] = pend_id[pl.ds(off + t * 16, 16)]
            ca = pltpu.async_copy(x_hbm.at[fire_idx], xbuf, sem1)
            cb = pltpu.async_copy(attn_hbm.at[fire_id], abuf, sem2)
            ca.wait()
            cb.wait()

            def edge(j):
                rowv = plsc.load_gather(pend_row, [_splat(off + j)])
                rbase = rowv * HD + iota
                jv = _splat(j)
                coefs = [plsc.load_gather(abuf, [jv, _splat(h)]) for h in range(H)]
                for k in range(8):
                    vals = xbuf[j, pl.ds(k * 16, 16)]
                    for h in range(H):
                        plsc.addupdate_scatter(
                            acc, [rbase + (h * D + k * 16)], vals * coefs[h])
            plsc.parallel_loop(0, FIRE, unroll=2)(edge)

        def scan_step(ci, cnt):
            base = ci * SCHUNK_C
            ca = pltpu.async_copy(src_hbm.at[pl.ds(base, SCHUNK_C)], srcb, sem1)
            cb = pltpu.async_copy(dst_hbm.at[pl.ds(base, SCHUNK_C)], dstb, sem2)
            ca.wait()
            cb.wait()

            def group(gi, cnt):
                sv = srcb[pl.ds(gi * 16, 16)]
                dv = dstb[pl.ds(gi * 16, 16)]
                ev = base + gi * 16 + iota
                m = (dv >= lo) & (dv < lo + R_C)
                cs = plsc.cumsum(jnp.where(m, 1, 0))
                pos = cnt + cs - 1
                plsc.store_scatter(pend_src, [pos], sv, mask=m)
                plsc.store_scatter(pend_row, [pos], dv - lo, mask=m)
                plsc.store_scatter(pend_id, [pos], ev, mask=m)
                return cnt + cs[15]

            cnt = plsc.parallel_loop(0, SCHUNK_C // 16, unroll=4, carry=cnt)(group)
            n_full = cnt // FIRE

            def fire_j(j, carry):
                do_fire(j * FIRE)
                return carry
            lax.fori_loop(0, n_full, fire_j, 0)

            ro = n_full * FIRE
            for t in range(8):
                pend_src[pl.ds(t * 16, 16)] = pend_src[pl.ds(ro + t * 16, 16)]
                pend_row[pl.ds(t * 16, 16)] = pend_row[pl.ds(ro + t * 16, 16)]
                pend_id[pl.ds(t * 16, 16)] = pend_id[pl.ds(ro + t * 16, 16)]
            return cnt - ro

        cnt = lax.fori_loop(0, E // SCHUNK_C, scan_step, jnp.int32(0))

        for t in range(8):
            pend_src[pl.ds(cnt + t * 16, 16)] = jnp.zeros((16,), jnp.int32)
            pend_row[pl.ds(cnt + t * 16, 16)] = jnp.full((16,), R_C, jnp.int32)
            pend_id[pl.ds(cnt + t * 16, 16)] = jnp.zeros((16,), jnp.int32)
        do_fire(0)

        pltpu.sync_copy(acc.at[pl.ds(0, R_C * HD)],
                        out_hbm.at[pl.ds(lo * HD, R_C * HD)])
        return pcarry

    lax.fori_loop(0, NPASS, pass_body, 0)


def _proj_body(x_ref, w_ref, att_ref, out_ref):
    w3 = w_ref[...].reshape(D, H, D)
    v = jnp.sum(w3 * att_ref[...][None, :, :], axis=-1)
    vpad = jnp.concatenate([v, jnp.zeros((D, 16 - H), jnp.float32)], axis=1)
    out_ref[...] = jnp.dot(x_ref[...], vpad, preferred_element_type=jnp.float32)


def _proj(x, W, att, blk):
    n = x.shape[0]
    return pl.pallas_call(
        _proj_body,
        grid=(n // blk,),
        in_specs=[
            pl.BlockSpec((blk, D), lambda i: (i, 0)),
            pl.BlockSpec((D, H * D), lambda i: (0, 0)),
            pl.BlockSpec((H, D), lambda i: (0, 0)),
        ],
        out_specs=pl.BlockSpec((blk, 16), lambda i: (i, 0)),
        out_shape=jax.ShapeDtypeStruct((n, 16), jnp.float32),
    )(x, W, att)


def _job_body(agg_ref, xj_ref, wrel_ref, wroot_ref, b_ref, out_ref):
    r = (jnp.dot(agg_ref[...], wrel_ref[...], preferred_element_type=jnp.float32)
         + jnp.dot(xj_ref[...], wroot_ref[...], preferred_element_type=jnp.float32)
         + b_ref[...])
    out_ref[...] = _leaky(r)


def _job_out(agg, x_job, W_rel, W_root, b_rel):
    blk = 2000
    return pl.pallas_call(
        _job_body,
        grid=(N_JOB // blk,),
        in_specs=[
            pl.BlockSpec((blk, D), lambda i: (i, 0)),
            pl.BlockSpec((blk, D), lambda i: (i, 0)),
            pl.BlockSpec((D, D), lambda i: (0, 0)),
            pl.BlockSpec((D, D), lambda i: (0, 0)),
            pl.BlockSpec((D,), lambda i: (0,)),
        ],
        out_specs=pl.BlockSpec((blk, D), lambda i: (i, 0)),
        out_shape=jax.ShapeDtypeStruct((N_JOB, D), jnp.float32),
    )(agg, x_job, W_rel, W_root, b_rel)


def _gatout_body(c_ref, ws_ref, b_ref, out_ref):
    r = jnp.dot(c_ref[...], ws_ref[...], preferred_element_type=jnp.float32)
    out_ref[...] = _leaky(r * (1.0 / H) + b_ref[...])


def _gat_out(C, W_stack, b_gat):
    blk = 2000
    return pl.pallas_call(
        _gatout_body,
        grid=(N_RES // blk,),
        in_specs=[
            pl.BlockSpec((blk, H * D), lambda i: (i, 0)),
            pl.BlockSpec((H * D, D), lambda i: (0, 0)),
            pl.BlockSpec((D,), lambda i: (0,)),
        ],
        out_specs=pl.BlockSpec((blk, D), lambda i: (i, 0)),
        out_shape=jax.ShapeDtypeStruct((N_RES, D), jnp.float32),
    )(C, W_stack, b_gat)


def _mlp_body(comb_ref, w1_ref, b1_ref, w2_ref, b2_ref, w3_ref, b3_ref, out_ref):
    comb = comb_ref[...]
    h1 = _leaky(jnp.dot(comb, w1_ref[...], preferred_element_type=jnp.float32) + b1_ref[...])
    h2 = _leaky(jnp.dot(h1, w2_ref[...], preferred_element_type=jnp.float32) + b2_ref[...])
    score = jnp.sum(h2 * w3_ref[...].reshape(1, D), axis=1) + b3_ref[0]
    out_ref[...] = score


def _mlp(comb, W1, b1, W2, b2, W3, b3):
    n = comb.shape[0]
    return pl.pallas_call(
        _mlp_body,
        out_shape=jax.ShapeDtypeStruct((n,), jnp.float32),
    )(comb, W1, b1, W2, b2, W3, b3)


def kernel(x_skill, x_job, x_resume, edge_index_sj, edge_index_sr, W_rel, b_rel, W_root, W_src, W_dst, att_src, att_dst, b_gat, W1, b1, W2, b2, W3, b3):
    src_j = edge_index_sj[0]
    dst_j = edge_index_sj[1]
    src_r = edge_index_sr[0]
    dst_r = edge_index_sr[1]

    agg = _gc_kernel(x_skill, src_j, dst_j).reshape(PAD_N, D)[:N_JOB]
    xj = _job_out(agg, x_job, W_rel, W_root, b_rel)

    a_src = _proj(x_skill, W_src, att_src, 2000)
    a_dst = _proj(x_resume, W_dst, att_dst, 2000)
    a_dst_pad = jnp.concatenate(
        [a_dst, jnp.zeros((PAD_N - N_RES, 16), jnp.float32)], axis=0)

    den = _den_kernel(a_src, a_dst_pad.reshape(-1), src_r, dst_r).reshape(PAD_N, 16)
    attn_pad = _attn_kernel(a_src, a_dst_pad, den, src_r, dst_r)
    attn = attn_pad[:, :H]

    C = _gatagg_kernel(x_skill, attn_pad, src_r, dst_r).reshape(PAD_N, H * D)[:N_RES]
    W_stack = W_src.reshape(D, H, D).transpose(1, 0, 2).reshape(H * D, D)
    xr = _gat_out(C, W_stack, b_gat)

    comb = jnp.concatenate([xj, xr], axis=1)
    score = _mlp(comb, W1, b1, W2, b2, W3, b3)
    return (score, attn)
